# Initial kernel scaffold; baseline (speedup 1.0000x reference)
import jax
import jax.numpy as jnp
from jax.experimental import pallas as pl


def kernel(H_rna, C, edge_attr_rna, im_w1, im_b1, im_w2, im_b2, im_uw, im_ub, mc_g1w, mc_g1b, mc_g2w, mc_g2b, mc_mw, mc_mb, mc_uw, mc_ub, cm_qw, cm_qb, cm_kw, cm_kb, cm_vw, cm_vb, cm_ow, cm_ob, cm_g1w, cm_g1b, cm_g2w, cm_g2b, cc_w1, cc_b1, cc_w2, cc_b2, cc_uw, cc_ub, edge_index_rna, edge_index_belong, edge_index_cell):
    raise NotImplementedError("write your pallas kernel here")


# speedup vs baseline: 1.4347x; 1.4347x over previous
import functools
import numpy as np
import jax
import jax.numpy as jnp
from jax import lax
from jax.experimental import pallas as pl
from jax.experimental.pallas import tpu as pltpu
from jax.experimental.pallas import tpu_sc as plsc

N = 10000
D = 128
E = 320000
NH, HD = 4, 32
NC, NS = 2, 16
NW = NC * NS
RW = E // NW
CH = 80
NCHK = RW // CH
PN = 10240
TR = PN // NS
CH3 = 40
F32 = jnp.float32

_MESH = plsc.VectorSubcoreMesh(
    core_axis_name="c", subcore_axis_name="s", num_cores=NC, num_subcores=NS)


def _sig(x):
    return 1.0 / (1.0 + jnp.exp(-x))


def _silu(x):
    return x * _sig(x)


def _zero_accum(zb, acc, sid, zr):
    def zrow(r, _):
        for q in range(D // 16):
            zb[r, pl.ds(q * 16, 16)] = jnp.zeros((16,), F32)
        return 0
    lax.fori_loop(0, zr, zrow, 0)
    def zcp(k, _):
        pltpu.sync_copy(zb, acc.at[pl.ds(sid * TR + k * zr, zr)])
        return 0
    lax.fori_loop(0, TR // zr, zcp, 0)


def _dump_accum(acc, outh, cid, sid, zr):
    def dcp(k, _):
        off = sid * TR + k * zr
        pltpu.sync_copy(acc.at[pl.ds(off, zr)],
                        outh.at[cid, pl.ds(off, zr)])
        return 0
    lax.fori_loop(0, TR // zr, dcp, 0)


def _ids():
    cid = lax.axis_index("c")
    sid = lax.axis_index("s")
    return cid, sid, sid * NC + cid


def _g2a_body(i1h, i2h, t1h, t2h, outh, i1v, i2v, b1v, b2v, s1, s2):
    _, _, wid = _ids()

    def step(j, _):
        base = wid * RW + j * CH
        pltpu.sync_copy(i1h.at[pl.ds(base, CH)], i1v)
        pltpu.sync_copy(i2h.at[pl.ds(base, CH)], i2v)
        d1 = pltpu.async_copy(t1h.at[i1v], b1v, s1)
        d2 = pltpu.async_copy(t2h.at[i2v], b2v, s2)
        d1.wait()
        d2.wait()

        def row(r, _):
            for q in range(D // 16):
                sl = pl.ds(q * 16, 16)
                b1v[r, sl] = b1v[r, sl] + b2v[r, sl]
            return 0
        lax.fori_loop(0, CH, row, 0)
        pltpu.sync_copy(b1v, outh.at[pl.ds(base, CH)])
        return 0
    lax.fori_loop(0, NCHK, step, 0)


_g2a = pl.kernel(
    _g2a_body,
    out_type=jax.ShapeDtypeStruct((E, D), F32),
    mesh=_MESH,
    scratch_types=[
        pltpu.VMEM((CH,), jnp.int32), pltpu.VMEM((CH,), jnp.int32),
        pltpu.VMEM((CH, D), F32), pltpu.VMEM((CH, D), F32),
        pltpu.SemaphoreType.DMA, pltpu.SemaphoreType.DMA,
    ],
)


def _g2_body(i1h, i2h, t1h, t2h, o1h, o2h, i1v, i2v, b1v, b2v, s1, s2):
    _, _, wid = _ids()

    def step(j, _):
        base = wid * RW + j * CH
        pltpu.sync_copy(i1h.at[pl.ds(base, CH)], i1v)
        pltpu.sync_copy(i2h.at[pl.ds(base, CH)], i2v)
        d1 = pltpu.async_copy(t1h.at[i1v], b1v, s1)
        d2 = pltpu.async_copy(t2h.at[i2v], b2v, s2)
        d1.wait()
        d2.wait()
        pltpu.sync_copy(b1v, o1h.at[pl.ds(base, CH)])
        pltpu.sync_copy(b2v, o2h.at[pl.ds(base, CH)])
        return 0
    lax.fori_loop(0, NCHK, step, 0)


_g2 = pl.kernel(
    _g2_body,
    out_type=(jax.ShapeDtypeStruct((E, D), F32),
              jax.ShapeDtypeStruct((E, D), F32)),
    mesh=_MESH,
    scratch_types=[
        pltpu.VMEM((CH,), jnp.int32), pltpu.VMEM((CH,), jnp.int32),
        pltpu.VMEM((CH, D), F32), pltpu.VMEM((CH, D), F32),
        pltpu.SemaphoreType.DMA, pltpu.SemaphoreType.DMA,
    ],
)


def _scat_body(idxh, mh, outh, iv, bm, zb, acc):
    cid, sid, wid = _ids()
    _zero_accum(zb, acc, sid, CH)
    plsc.subcore_barrier()

    def step(j, _):
        base = wid * RW + j * CH
        pltpu.sync_copy(idxh.at[pl.ds(base, CH)], iv)
        pltpu.sync_copy(mh.at[pl.ds(base, CH)], bm)
        pltpu.sync_copy(bm, acc.at[iv], add=True)
        return 0
    lax.fori_loop(0, NCHK, step, 0)
    plsc.subcore_barrier()
    _dump_accum(acc, outh, cid, sid, CH)


_scat = pl.kernel(
    _scat_body,
    out_type=jax.ShapeDtypeStruct((2, PN, D), F32),
    mesh=_MESH,
    scratch_types=[
        pltpu.VMEM((CH,), jnp.int32), pltpu.VMEM((CH, D), F32),
        pltpu.VMEM((CH, D), F32), pltpu.VMEM_SHARED((PN, D), F32),
    ],
)


def _s2_body(bsh, bdh, xph, cph, outh, ibv, idv, bx, bc, s1, s2, zb, acc):
    cid, sid, wid = _ids()
    _zero_accum(zb, acc, sid, CH)
    plsc.subcore_barrier()

    def step(j, _):
        base = wid * RW + j * CH
        pltpu.sync_copy(bsh.at[pl.ds(base, CH)], ibv)
        pltpu.sync_copy(bdh.at[pl.ds(base, CH)], idv)
        d1 = pltpu.async_copy(xph.at[ibv], bx, s1)
        d2 = pltpu.async_copy(cph.at[idv], bc, s2)
        d1.wait()
        d2.wait()

        def row(r, _):
            for q in range(D // 16):
                sl = pl.ds(q * 16, 16)
                x = bx[r, sl] + bc[r, sl]
                x = x * (1.0 / (1.0 + jnp.exp(-x)))
                bx[r, sl] = x * (1.0 / (1.0 + jnp.exp(-x)))
            return 0
        lax.fori_loop(0, CH, row, 0)
        pltpu.sync_copy(bx, acc.at[idv], add=True)
        return 0
    lax.fori_loop(0, NCHK, step, 0)
    plsc.subcore_barrier()
    _dump_accum(acc, outh, cid, sid, CH)


_s2 = pl.kernel(
    _s2_body,
    out_type=jax.ShapeDtypeStruct((2, PN, D), F32),
    mesh=_MESH,
    scratch_types=[
        pltpu.VMEM((CH,), jnp.int32), pltpu.VMEM((CH,), jnp.int32),
        pltpu.VMEM((CH, D), F32), pltpu.VMEM((CH, D), F32),
        pltpu.SemaphoreType.DMA, pltpu.SemaphoreType.DMA,
        pltpu.VMEM((CH, D), F32), pltpu.VMEM_SHARED((PN, D), F32),
    ],
)


def _s3_body(bsh, bdh, voh, wh, outh, ibv, idv, bvo, bw, bm, s1, zb, acc):
    cid, sid, wid = _ids()
    _zero_accum(zb, acc, sid, CH3)
    plsc.subcore_barrier()

    def step(j, _):
        base = wid * RW + j * CH3
        pltpu.sync_copy(bsh.at[pl.ds(base, CH3)], ibv)
        pltpu.sync_copy(bdh.at[pl.ds(base, CH3)], idv)
        pltpu.sync_copy(wh.at[pl.ds(base, CH3)], bw)
        pltpu.async_copy(voh.at[idv], bvo, s1).wait()

        def row(r, _):
            w0 = bw[r, pl.ds(0, 16)]
            w1 = bw[r, pl.ds(32, 16)]
            w2 = bw[r, pl.ds(64, 16)]
            w3 = bw[r, pl.ds(96, 16)]
            for q in range(D // 16):
                o = (w0 * bvo[r, pl.ds(q * 16, 16)]
                     + w1 * bvo[r, pl.ds(D + q * 16, 16)]
                     + w2 * bvo[r, pl.ds(2 * D + q * 16, 16)]
                     + w3 * bvo[r, pl.ds(3 * D + q * 16, 16)])
                bm[r, pl.ds(q * 16, 16)] = o * (1.0 / (1.0 + jnp.exp(-o)))
            return 0
        lax.fori_loop(0, CH3, row, 0)
        pltpu.sync_copy(bm, acc.at[ibv], add=True)
        return 0
    lax.fori_loop(0, RW // CH3, step, 0)
    plsc.subcore_barrier()
    _dump_accum(acc, outh, cid, sid, CH3)


_s3 = pl.kernel(
    _s3_body,
    out_type=jax.ShapeDtypeStruct((2, PN, D), F32),
    mesh=_MESH,
    scratch_types=[
        pltpu.VMEM((CH3,), jnp.int32), pltpu.VMEM((CH3,), jnp.int32),
        pltpu.VMEM((CH3, NH * D), F32), pltpu.VMEM((CH3, D), F32),
        pltpu.VMEM((CH3, D), F32),
        pltpu.SemaphoreType.DMA,
        pltpu.VMEM((CH3, D), F32), pltpu.VMEM_SHARED((PN, D), F32),
    ],
)


_RB = 1000
_EB = 2000


def _nspec(cols=D):
    return pl.BlockSpec((_RB, cols), lambda i: (i, 0))


def _wspec(r, c):
    return pl.BlockSpec((r, c), lambda i: (0, 0))


def _espec(cols=D):
    return pl.BlockSpec((_EB, cols), lambda i: (i, 0))


def _tc_nodes1(h, w1dT, w1sT, b1):
    def body(h_ref, wd_ref, ws_ref, b1_ref, a_ref, b_ref):
        hh = h_ref[...]
        a_ref[...] = jnp.dot(hh, wd_ref[...], preferred_element_type=F32)
        b_ref[...] = jnp.dot(hh, ws_ref[...], preferred_element_type=F32) + b1_ref[...]
    return pl.pallas_call(
        body, grid=(N // _RB,),
        in_specs=[_nspec(), _wspec(D, D), _wspec(D, D), _wspec(1, D)],
        out_specs=[_nspec(), _nspec()],
        out_shape=[jax.ShapeDtypeStruct((N, D), F32)] * 2,
    )(h, w1dT, w1sT, b1)


def _tc_edge_mlp(g, ea, w1eT, w2T, b2):
    if ea is None:
        def body(g_ref, w2_ref, b2_ref, m_ref):
            m = _silu(g_ref[...])
            m = _silu(jnp.dot(m, w2_ref[...], preferred_element_type=F32)
                      + b2_ref[...])
            m_ref[...] = _silu(m)
        return pl.pallas_call(
            body, grid=(E // _EB,),
            in_specs=[_espec(), _wspec(D, D), _wspec(1, D)],
            out_specs=_espec(),
            out_shape=jax.ShapeDtypeStruct((E, D), F32),
        )(g, w2T, b2)

    def body(g_ref, ea_ref, w1e_ref, w2_ref, b2_ref, m_ref):
        pre = g_ref[...] + jnp.dot(ea_ref[...], w1e_ref[...],
                                   preferred_element_type=F32)
        m = _silu(pre)
        m = _silu(jnp.dot(m, w2_ref[...], preferred_element_type=F32)
                  + b2_ref[...])
        m_ref[...] = _silu(m)
    return pl.pallas_call(
        body, grid=(E // _EB,),
        in_specs=[_espec(), _espec(16), _wspec(16, D), _wspec(D, D),
                  _wspec(1, D)],
        out_specs=_espec(),
        out_shape=jax.ShapeDtypeStruct((E, D), F32),
    )(g, ea, w1eT, w2T, b2)


def _pspec(core):
    return pl.BlockSpec((1, _RB, D), lambda i, c=core: (c, i, 0))


def _tc_update(parts, x, uwT, ub):
    def body(p0_ref, p1_ref, x_ref, uw_ref, ub_ref, o_ref):
        s = p0_ref[0] + p1_ref[0]
        o_ref[...] = (x_ref[...]
                      + jnp.dot(s, uw_ref[...], preferred_element_type=F32)
                      + ub_ref[...])
    return pl.pallas_call(
        body, grid=(N // _RB,),
        in_specs=[_pspec(0), _pspec(1),
                  _nspec(), _wspec(D, D), _wspec(1, D)],
        out_specs=_nspec(),
        out_shape=jax.ShapeDtypeStruct((N, D), F32),
    )(parts, parts, x, uwT, ub)


def _tc_nodes2(h1, c, mwxT, mwcT, mb, g1wT, g1b, g2w, g2b):
    def body(h1_ref, c_ref, wx_ref, wc_ref, mb_ref, g1_ref, g1b_ref,
             g2_ref, g2b_ref, xp_ref, cp_ref, a16_ref):
        cc = c_ref[...]
        xp_ref[...] = jnp.dot(h1_ref[...], wx_ref[...],
                              preferred_element_type=F32)
        cp_ref[...] = jnp.dot(cc, wc_ref[...],
                              preferred_element_type=F32) + mb_ref[...]
        t = _silu(jnp.dot(cc, g1_ref[...], preferred_element_type=F32)
                  + g1b_ref[...])
        a = _sig(jnp.sum(t * g2_ref[...], axis=1, keepdims=True)
                 + g2b_ref[...])
        a16_ref[...] = jnp.broadcast_to(a, (_RB, 16))
    return pl.pallas_call(
        body, grid=(N // _RB,),
        in_specs=[_nspec(), _nspec(), _wspec(D, D), _wspec(D, D),
                  _wspec(1, D), _wspec(D, D), _wspec(1, D), _wspec(1, D),
                  _wspec(1, 1)],
        out_specs=[_nspec(), _nspec(), _nspec(16)],
        out_shape=[jax.ShapeDtypeStruct((N, D), F32),
                   jax.ShapeDtypeStruct((N, D), F32),
                   jax.ShapeDtypeStruct((N, 16), F32)],
    )(h1, c, mwxT, mwcT, mb, g1wT, g1b, g2w, g2b)


def _tc_nodes3(h1, c1, qwT, qb, kwT, kb, vwT, vb, bd, obt, g1wT, g1b,
               g2w, g2b):
    def body(h1_ref, c1_ref, qw_ref, qb_ref, kw_ref, kb_ref, vw_ref,
             vb_ref, bd_ref, obt_ref, g1_ref, g1b_ref, g2_ref, g2b_ref,
             q_ref, k_ref, vo_ref, g16_ref):
        hh = h1_ref[...]
        cc = c1_ref[...]
        q_ref[...] = jnp.dot(hh, qw_ref[...], preferred_element_type=F32) + qb_ref[...]
        k_ref[...] = jnp.dot(cc, kw_ref[...], preferred_element_type=F32) + kb_ref[...]
        v = jnp.dot(cc, vw_ref[...], preferred_element_type=F32) + vb_ref[...]
        vo_ref[...] = jnp.dot(v, bd_ref[...], preferred_element_type=F32) + obt_ref[...]
        t = _silu(jnp.dot(hh, g1_ref[...], preferred_element_type=F32)
                  + g1b_ref[...])
        g = _sig(jnp.sum(t * g2_ref[...], axis=1, keepdims=True)
                 + g2b_ref[...])
        g16_ref[...] = jnp.broadcast_to(g, (_RB, 16))
    return pl.pallas_call(
        body, grid=(N // _RB,),
        in_specs=[_nspec(), _nspec(), _wspec(D, D), _wspec(1, D),
                  _wspec(D, D), _wspec(1, D), _wspec(D, D), _wspec(1, D),
                  _wspec(D, NH * D), _wspec(1, NH * D), _wspec(D, D),
                  _wspec(1, D), _wspec(1, D), _wspec(1, 1)],
        out_specs=[_nspec(), _nspec(), _nspec(NH * D), _nspec(16)],
        out_shape=[jax.ShapeDtypeStruct((N, D), F32),
                   jax.ShapeDtypeStruct((N, D), F32),
                   jax.ShapeDtypeStruct((N, NH * D), F32),
                   jax.ShapeDtypeStruct((N, 16), F32)],
    )(h1, c1, qwT, qb, kwT, kb, vwT, vb, bd, obt, g1wT, g1b, g2w, g2b)


def _tc_scores(gq, gk):
    inv = float(1.0 / np.sqrt(HD))

    def body(q_ref, k_ref, w_ref):
        p = q_ref[...] * k_ref[...]
        s = [jnp.sum(p[:, HD * h:HD * (h + 1)], axis=1, keepdims=True) * inv
             for h in range(NH)]
        m = jnp.maximum(jnp.maximum(s[0], s[1]), jnp.maximum(s[2], s[3]))
        e = [jnp.exp(sh - m) for sh in s]
        z = e[0] + e[1] + e[2] + e[3]
        w_ref[...] = jnp.concatenate(
            [jnp.broadcast_to(eh / z, (_EB, HD)) for eh in e], axis=1)
    return pl.pallas_call(
        body, grid=(E // _EB,),
        in_specs=[_espec(), _espec()],
        out_specs=_espec(),
        out_shape=jax.ShapeDtypeStruct((E, D), F32),
    )(gq, gk)


def _tc_update_gated(parts, x, uwT, ub, a16):
    def body(p0_ref, p1_ref, x_ref, uw_ref, ub_ref, a_ref, o_ref):
        s = (p0_ref[0] + p1_ref[0]) * a_ref[:, 0:1]
        o_ref[...] = (x_ref[...]
                      + jnp.dot(s, uw_ref[...], preferred_element_type=F32)
                      + ub_ref[...])
    return pl.pallas_call(
        body, grid=(N // _RB,),
        in_specs=[_pspec(0), _pspec(1), _nspec(), _wspec(D, D),
                  _wspec(1, D), _nspec(16)],
        out_specs=_nspec(),
        out_shape=jax.ShapeDtypeStruct((N, D), F32),
    )(parts, parts, x, uwT, ub, a16)


def _tc_add3_gated(parts, x, skip, g16):
    def body(p0_ref, p1_ref, x_ref, s_ref, g_ref, o_ref):
        o_ref[...] = ((p0_ref[0] + p1_ref[0]) * g_ref[:, 0:1]
                      + x_ref[...] + s_ref[...])
    return pl.pallas_call(
        body, grid=(N // _RB,),
        in_specs=[_pspec(0), _pspec(1), _nspec(), _nspec(), _nspec(16)],
        out_specs=_nspec(),
        out_shape=jax.ShapeDtypeStruct((N, D), F32),
    )(parts, parts, x, skip, g16)


def _tc_update_skip(parts, x, uwT, ub, skip):
    def body(p0_ref, p1_ref, x_ref, uw_ref, ub_ref, s_ref, o_ref):
        s = p0_ref[0] + p1_ref[0]
        o_ref[...] = (x_ref[...]
                      + jnp.dot(s, uw_ref[...], preferred_element_type=F32)
                      + ub_ref[...] + s_ref[...])
    return pl.pallas_call(
        body, grid=(N // _RB,),
        in_specs=[_pspec(0), _pspec(1), _nspec(), _wspec(D, D),
                  _wspec(1, D), _nspec()],
        out_specs=_nspec(),
        out_shape=jax.ShapeDtypeStruct((N, D), F32),
    )(parts, parts, x, uwT, ub, skip)


def kernel(H_rna, C, edge_attr_rna, im_w1, im_b1, im_w2, im_b2, im_uw,
           im_ub, mc_g1w, mc_g1b, mc_g2w, mc_g2b, mc_mw, mc_mb, mc_uw,
           mc_ub, cm_qw, cm_qb, cm_kw, cm_kb, cm_vw, cm_vb, cm_ow, cm_ob,
           cm_g1w, cm_g1b, cm_g2w, cm_g2b, cc_w1, cc_b1, cc_w2, cc_b2,
           cc_uw, cc_ub, edge_index_rna, edge_index_belong,
           edge_index_cell):
    src, dst = edge_index_rna[0], edge_index_rna[1]
    bsrc, bdst = edge_index_belong[0], edge_index_belong[1]
    csrc, cdst = edge_index_cell[0], edge_index_cell[1]
    row = lambda b: b.reshape(1, -1)

    A, B = _tc_nodes1(H_rna, im_w1[:, :D].T, im_w1[:, D:2 * D].T,
                      row(im_b1))
    G1 = _g2a(dst, src, A, B)
    M1 = _tc_edge_mlp(G1, edge_attr_rna, im_w1[:, 2 * D:].T, im_w2.T,
                      row(im_b2))
    P1 = _scat(dst, M1)
    H1 = _tc_update(P1, H_rna, im_uw.T, row(im_ub))

    Xp, Cp, A16 = _tc_nodes2(H1, C, mc_mw[:, :D].T, mc_mw[:, D:].T,
                             row(mc_mb), mc_g1w.T, row(mc_g1b), mc_g2w,
                             mc_g2b.reshape(1, 1))
    P2 = _s2(bsrc, bdst, Xp, Cp)
    C1 = _tc_update_gated(P2, C, mc_uw.T, row(mc_ub), A16)

    bdm = jnp.zeros((D, NH * D), F32)
    for h in range(NH):
        bdm = bdm.at[HD * h:HD * (h + 1), D * h:D * (h + 1)].set(
            cm_ow[:, HD * h:HD * (h + 1)].T)
    obt = jnp.tile(cm_ob, NH).reshape(1, NH * D)
    Qn, Kn, VO, G16 = _tc_nodes3(H1, C1, cm_qw.T, row(cm_qb), cm_kw.T,
                                 row(cm_kb), cm_vw.T, row(cm_vb), bdm,
                                 obt, cm_g1w.T, row(cm_g1b), cm_g2w,
                                 cm_g2b.reshape(1, 1))
    GQ, GK = _g2(bsrc, bdst, Qn, Kn)
    W = _tc_scores(GQ, GK)
    P3 = _s3(bsrc, bdst, VO, W)
    Hout = _tc_add3_gated(P3, H1, H_rna, G16)

    A2, B2 = _tc_nodes1(C1, cc_w1[:, :D].T, cc_w1[:, D:].T, row(cc_b1))
    G4 = _g2a(cdst, csrc, A2, B2)
    M4 = _tc_edge_mlp(G4, None, None, cc_w2.T, row(cc_b2))
    P4 = _scat(cdst, M4)
    Cout = _tc_update_skip(P4, C1, cc_uw.T, row(cc_ub), C)

    return (Hout, Cout)


# speedup vs baseline: 3.1735x; 2.2120x over previous
import functools
import numpy as np
import jax
import jax.numpy as jnp
from jax import lax
from jax.experimental import pallas as pl
from jax.experimental.pallas import tpu as pltpu
from jax.experimental.pallas import tpu_sc as plsc

N = 10000
D = 128
E = 320000
NH, HD = 4, 32
NC, NS = 2, 16
NW = NC * NS
RW = E // NW
CH = 80
NCHK = RW // CH
PN = 10240
TR = PN // NS
F32 = jnp.float32

_MESH = plsc.VectorSubcoreMesh(
    core_axis_name="c", subcore_axis_name="s", num_cores=NC, num_subcores=NS)


def _sig(x):
    return 1.0 / (1.0 + jnp.exp(-x))


def _silu(x):
    return x * _sig(x)


def _ids():
    cid = lax.axis_index("c")
    sid = lax.axis_index("s")
    return cid, sid, sid * NC + cid


def _zero_accum(zb, acc, sid):
    def zrow(r, _):
        for q in range(D // 16):
            zb[r, pl.ds(q * 16, 16)] = jnp.zeros((16,), F32)
        return 0
    lax.fori_loop(0, CH, zrow, 0)

    def zcp(k, _):
        pltpu.sync_copy(zb, acc.at[pl.ds(sid * TR + k * CH, CH)])
        return 0
    lax.fori_loop(0, TR // CH, zcp, 0)


def _dump_accum(acc, outh, cid, sid):
    def dcp(k, _):
        off = sid * TR + k * CH
        pltpu.sync_copy(acc.at[pl.ds(off, CH)],
                        outh.at[cid, pl.ds(off, CH)])
        return 0
    lax.fori_loop(0, TR // CH, dcp, 0)


def _pipe(prime, work):
    prime(0, 0)

    def pair(g, _):
        for b in (0, 1):
            j = 2 * g + b
            prime(j + 1, b ^ 1)
            work(j, b)
        return 0
    lax.fori_loop(0, (NCHK - 1) // 2, pair, 0)
    work(NCHK - 1, (NCHK - 1) % 2)


def _g2a_body(i1h, i2h, t1h, t2h, outh,
              i1v0, i1v1, i2v0, i2v1, b10, b11, b20, b21, sa0, sa1, sb0, sb1):
    _, _, wid = _ids()
    i1v, i2v = (i1v0, i1v1), (i2v0, i2v1)
    b1v, b2v = (b10, b11), (b20, b21)
    sa, sb = (sa0, sa1), (sb0, sb1)

    def prime(j, b):
        base = wid * RW + j * CH
        pltpu.sync_copy(i1h.at[pl.ds(base, CH)], i1v[b])
        pltpu.sync_copy(i2h.at[pl.ds(base, CH)], i2v[b])
        pltpu.async_copy(t1h.at[i1v[b]], b1v[b], sa[b])
        pltpu.async_copy(t2h.at[i2v[b]], b2v[b], sb[b])

    def work(j, b):
        base = wid * RW + j * CH
        pltpu.make_async_copy(t1h.at[i1v[b]], b1v[b], sa[b]).wait()
        pltpu.make_async_copy(t2h.at[i2v[b]], b2v[b], sb[b]).wait()

        def row(r, _):
            for q in range(D // 16):
                sl = pl.ds(q * 16, 16)
                b1v[b][r, sl] = b1v[b][r, sl] + b2v[b][r, sl]
            return 0
        lax.fori_loop(0, CH, row, 0)
        pltpu.sync_copy(b1v[b], outh.at[pl.ds(base, CH)])

    _pipe(prime, work)


_g2a = pl.kernel(
    _g2a_body,
    out_type=jax.ShapeDtypeStruct((E, D), F32),
    mesh=_MESH,
    scratch_types=[
        pltpu.VMEM((CH,), jnp.int32), pltpu.VMEM((CH,), jnp.int32),
        pltpu.VMEM((CH,), jnp.int32), pltpu.VMEM((CH,), jnp.int32),
        pltpu.VMEM((CH, D), F32), pltpu.VMEM((CH, D), F32),
        pltpu.VMEM((CH, D), F32), pltpu.VMEM((CH, D), F32),
        pltpu.SemaphoreType.DMA, pltpu.SemaphoreType.DMA,
        pltpu.SemaphoreType.DMA, pltpu.SemaphoreType.DMA,
    ],
)


def _g3_body(ibh, idh, qth, kth, vth, oqh, okh, ovh,
             ib0, ib1, id0, id1, bq0, bq1, bk0, bk1, bv0, bv1,
             sq0, sq1, sk0, sk1, sv0, sv1):
    _, _, wid = _ids()
    ibv, idv = (ib0, ib1), (id0, id1)
    bq, bk, bv = (bq0, bq1), (bk0, bk1), (bv0, bv1)
    sq, sk, sv = (sq0, sq1), (sk0, sk1), (sv0, sv1)

    def prime(j, b):
        base = wid * RW + j * CH
        pltpu.sync_copy(ibh.at[pl.ds(base, CH)], ibv[b])
        pltpu.sync_copy(idh.at[pl.ds(base, CH)], idv[b])
        pltpu.async_copy(qth.at[ibv[b]], bq[b], sq[b])
        pltpu.async_copy(kth.at[idv[b]], bk[b], sk[b])
        pltpu.async_copy(vth.at[idv[b]], bv[b], sv[b])

    def work(j, b):
        base = wid * RW + j * CH
        pltpu.make_async_copy(qth.at[ibv[b]], bq[b], sq[b]).wait()
        pltpu.make_async_copy(kth.at[idv[b]], bk[b], sk[b]).wait()
        pltpu.make_async_copy(vth.at[idv[b]], bv[b], sv[b]).wait()
        pltpu.sync_copy(bq[b], oqh.at[pl.ds(base, CH)])
        pltpu.sync_copy(bk[b], okh.at[pl.ds(base, CH)])
        pltpu.sync_copy(bv[b], ovh.at[pl.ds(base, CH)])

    _pipe(prime, work)


_g3 = pl.kernel(
    _g3_body,
    out_type=(jax.ShapeDtypeStruct((E, D), F32),
              jax.ShapeDtypeStruct((E, D), F32),
              jax.ShapeDtypeStruct((E, D), F32)),
    mesh=_MESH,
    scratch_types=(
        [pltpu.VMEM((CH,), jnp.int32)] * 4
        + [pltpu.VMEM((CH, D), F32)] * 6
        + [pltpu.SemaphoreType.DMA] * 6
    ),
)


def _make_scat():
    def body(idxh, mh, outh, iv0, iv1, bm0, bm1, s0, s1, acc):
        cid, sid, wid = _ids()
        iv, bm, sm = (iv0, iv1), (bm0, bm1), (s0, s1)
        _zero_accum(bm0, acc, sid)
        plsc.subcore_barrier()

        def prime(j, b):
            base = wid * RW + j * CH
            pltpu.sync_copy(idxh.at[pl.ds(base, CH)], iv[b])
            pltpu.async_copy(mh.at[pl.ds(base, CH)], bm[b], sm[b])

        def work(j, b):
            base = wid * RW + j * CH
            pltpu.make_async_copy(mh.at[pl.ds(base, CH)], bm[b], sm[b]).wait()
            pltpu.sync_copy(bm[b], acc.at[iv[b]], add=True)

        _pipe(prime, work)
        plsc.subcore_barrier()
        _dump_accum(acc, outh, cid, sid)

    return pl.kernel(
        body,
        out_type=jax.ShapeDtypeStruct((2, PN, D), F32),
        mesh=_MESH,
        scratch_types=[
            pltpu.VMEM((CH,), jnp.int32), pltpu.VMEM((CH,), jnp.int32),
            pltpu.VMEM((CH, D), F32), pltpu.VMEM((CH, D), F32),
            pltpu.SemaphoreType.DMA, pltpu.SemaphoreType.DMA,
            pltpu.VMEM_SHARED((PN, D), F32),
        ],
    )


_scat = _make_scat()


def _make_s2():
    def body(bsh, bdh, xph, cph, outh,
             ib0, ib1, id0, id1, bx0, bx1, bc0, bc1, sa0, sa1, sb0, sb1,
             acc):
        cid, sid, wid = _ids()
        ibv, idv = (ib0, ib1), (id0, id1)
        bx, bc = (bx0, bx1), (bc0, bc1)
        sa, sb = (sa0, sa1), (sb0, sb1)
        _zero_accum(bx0, acc, sid)
        plsc.subcore_barrier()

        def prime(j, b):
            base = wid * RW + j * CH
            pltpu.sync_copy(bsh.at[pl.ds(base, CH)], ibv[b])
            pltpu.sync_copy(bdh.at[pl.ds(base, CH)], idv[b])
            pltpu.async_copy(xph.at[ibv[b]], bx[b], sa[b])
            pltpu.async_copy(cph.at[idv[b]], bc[b], sb[b])

        def work(j, b):
            pltpu.make_async_copy(xph.at[ibv[b]], bx[b], sa[b]).wait()
            pltpu.make_async_copy(cph.at[idv[b]], bc[b], sb[b]).wait()

            def row(r, _):
                for q in range(D // 16):
                    sl = pl.ds(q * 16, 16)
                    x = bx[b][r, sl] + bc[b][r, sl]
                    x = x * (1.0 / (1.0 + jnp.exp(-x)))
                    bx[b][r, sl] = x * (1.0 / (1.0 + jnp.exp(-x)))
                return 0
            lax.fori_loop(0, CH, row, 0)
            pltpu.sync_copy(bx[b], acc.at[idv[b]], add=True)

        _pipe(prime, work)
        plsc.subcore_barrier()
        _dump_accum(acc, outh, cid, sid)

    return pl.kernel(
        body,
        out_type=jax.ShapeDtypeStruct((2, PN, D), F32),
        mesh=_MESH,
        scratch_types=(
            [pltpu.VMEM((CH,), jnp.int32)] * 4
            + [pltpu.VMEM((CH, D), F32)] * 4
            + [pltpu.SemaphoreType.DMA] * 4
            + [pltpu.VMEM_SHARED((PN, D), F32)]
        ),
    )


_s2 = _make_s2()


_RB = 1000
_EB = 2000


def _nspec(cols=D):
    return pl.BlockSpec((_RB, cols), lambda i: (i, 0))


def _wspec(r, c):
    return pl.BlockSpec((r, c), lambda i: (0, 0))


def _espec(cols=D):
    return pl.BlockSpec((_EB, cols), lambda i: (i, 0))


def _pspec(core):
    return pl.BlockSpec((1, _RB, D), lambda i, c=core: (c, i, 0))


def _tc_nodes1(h, w1dT, w1sT, b1):
    def body(h_ref, wd_ref, ws_ref, b1_ref, a_ref, b_ref):
        hh = h_ref[...]
        a_ref[...] = jnp.dot(hh, wd_ref[...], preferred_element_type=F32)
        b_ref[...] = jnp.dot(hh, ws_ref[...], preferred_element_type=F32) + b1_ref[...]
    return pl.pallas_call(
        body, grid=(N // _RB,),
        in_specs=[_nspec(), _wspec(D, D), _wspec(D, D), _wspec(1, D)],
        out_specs=[_nspec(), _nspec()],
        out_shape=[jax.ShapeDtypeStruct((N, D), F32)] * 2,
    )(h, w1dT, w1sT, b1)


def _tc_edge_mlp(g, ea, w1eT, w2T, b2):
    if ea is None:
        def body(g_ref, w2_ref, b2_ref, m_ref):
            m = _silu(g_ref[...])
            m = _silu(jnp.dot(m, w2_ref[...], preferred_element_type=F32)
                      + b2_ref[...])
            m_ref[...] = _silu(m)
        return pl.pallas_call(
            body, grid=(E // _EB,),
            in_specs=[_espec(), _wspec(D, D), _wspec(1, D)],
            out_specs=_espec(),
            out_shape=jax.ShapeDtypeStruct((E, D), F32),
        )(g, w2T, b2)

    def body(g_ref, ea_ref, w1e_ref, w2_ref, b2_ref, m_ref):
        pre = g_ref[...] + jnp.dot(ea_ref[...], w1e_ref[...],
                                   preferred_element_type=F32)
        m = _silu(pre)
        m = _silu(jnp.dot(m, w2_ref[...], preferred_element_type=F32)
                  + b2_ref[...])
        m_ref[...] = _silu(m)
    return pl.pallas_call(
        body, grid=(E // _EB,),
        in_specs=[_espec(), _espec(16), _wspec(16, D), _wspec(D, D),
                  _wspec(1, D)],
        out_specs=_espec(),
        out_shape=jax.ShapeDtypeStruct((E, D), F32),
    )(g, ea, w1eT, w2T, b2)


def _tc_update(parts, x, uwT, ub):
    def body(p0_ref, p1_ref, x_ref, uw_ref, ub_ref, o_ref):
        s = p0_ref[0] + p1_ref[0]
        o_ref[...] = (x_ref[...]
                      + jnp.dot(s, uw_ref[...], preferred_element_type=F32)
                      + ub_ref[...])
    return pl.pallas_call(
        body, grid=(N // _RB,),
        in_specs=[_pspec(0), _pspec(1),
                  _nspec(), _wspec(D, D), _wspec(1, D)],
        out_specs=_nspec(),
        out_shape=jax.ShapeDtypeStruct((N, D), F32),
    )(parts, parts, x, uwT, ub)


def _tc_nodes2(h1, c, mwxT, mwcT, mb, g1wT, g1b, g2w, g2b):
    def body(h1_ref, c_ref, wx_ref, wc_ref, mb_ref, g1_ref, g1b_ref,
             g2_ref, g2b_ref, xp_ref, cp_ref, a16_ref):
        cc = c_ref[...]
        xp_ref[...] = jnp.dot(h1_ref[...], wx_ref[...],
                              preferred_element_type=F32)
        cp_ref[...] = jnp.dot(cc, wc_ref[...],
                              preferred_element_type=F32) + mb_ref[...]
        t = _silu(jnp.dot(cc, g1_ref[...], preferred_element_type=F32)
                  + g1b_ref[...])
        a = _sig(jnp.sum(t * g2_ref[...], axis=1, keepdims=True)
                 + g2b_ref[...])
        a16_ref[...] = jnp.broadcast_to(a, (_RB, 16))
    return pl.pallas_call(
        body, grid=(N // _RB,),
        in_specs=[_nspec(), _nspec(), _wspec(D, D), _wspec(D, D),
                  _wspec(1, D), _wspec(D, D), _wspec(1, D), _wspec(1, D),
                  _wspec(1, 1)],
        out_specs=[_nspec(), _nspec(), _nspec(16)],
        out_shape=[jax.ShapeDtypeStruct((N, D), F32),
                   jax.ShapeDtypeStruct((N, D), F32),
                   jax.ShapeDtypeStruct((N, 16), F32)],
    )(h1, c, mwxT, mwcT, mb, g1wT, g1b, g2w, g2b)


def _tc_update_gated(parts, x, uwT, ub, a16):
    def body(p0_ref, p1_ref, x_ref, uw_ref, ub_ref, a_ref, o_ref):
        s = (p0_ref[0] + p1_ref[0]) * a_ref[:, 0:1]
        o_ref[...] = (x_ref[...]
                      + jnp.dot(s, uw_ref[...], preferred_element_type=F32)
                      + ub_ref[...])
    return pl.pallas_call(
        body, grid=(N // _RB,),
        in_specs=[_pspec(0), _pspec(1), _nspec(), _wspec(D, D),
                  _wspec(1, D), _nspec(16)],
        out_specs=_nspec(),
        out_shape=jax.ShapeDtypeStruct((N, D), F32),
    )(parts, parts, x, uwT, ub, a16)


def _tc_add3_gated(parts, x, skip, g16):
    def body(p0_ref, p1_ref, x_ref, s_ref, g_ref, o_ref):
        o_ref[...] = ((p0_ref[0] + p1_ref[0]) * g_ref[:, 0:1]
                      + x_ref[...] + s_ref[...])
    return pl.pallas_call(
        body, grid=(N // _RB,),
        in_specs=[_pspec(0), _pspec(1), _nspec(), _nspec(), _nspec(16)],
        out_specs=_nspec(),
        out_shape=jax.ShapeDtypeStruct((N, D), F32),
    )(parts, parts, x, skip, g16)


def _tc_nodes3(h1, c1, qwT, qb, kwT, kb, vwT, vb, g1wT, g1b, g2w, g2b):
    def body(h1_ref, c1_ref, qw_ref, qb_ref, kw_ref, kb_ref, vw_ref,
             vb_ref, g1_ref, g1b_ref, g2_ref, g2b_ref,
             q_ref, k_ref, v_ref, g16_ref):
        hh = h1_ref[...]
        cc = c1_ref[...]
        q_ref[...] = jnp.dot(hh, qw_ref[...], preferred_element_type=F32) + qb_ref[...]
        k_ref[...] = jnp.dot(cc, kw_ref[...], preferred_element_type=F32) + kb_ref[...]
        v_ref[...] = jnp.dot(cc, vw_ref[...], preferred_element_type=F32) + vb_ref[...]
        t = _silu(jnp.dot(hh, g1_ref[...], preferred_element_type=F32)
                  + g1b_ref[...])
        g = _sig(jnp.sum(t * g2_ref[...], axis=1, keepdims=True)
                 + g2b_ref[...])
        g16_ref[...] = jnp.broadcast_to(g, (_RB, 16))
    return pl.pallas_call(
        body, grid=(N // _RB,),
        in_specs=[_nspec(), _nspec(), _wspec(D, D), _wspec(1, D),
                  _wspec(D, D), _wspec(1, D), _wspec(D, D), _wspec(1, D),
                  _wspec(D, D), _wspec(1, D), _wspec(1, D), _wspec(1, 1)],
        out_specs=[_nspec(), _nspec(), _nspec(), _nspec(16)],
        out_shape=[jax.ShapeDtypeStruct((N, D), F32),
                   jax.ShapeDtypeStruct((N, D), F32),
                   jax.ShapeDtypeStruct((N, D), F32),
                   jax.ShapeDtypeStruct((N, 16), F32)],
    )(h1, c1, qwT, qb, kwT, kb, vwT, vb, g1wT, g1b, g2w, g2b)


def _tc_attn(gq, gk, gv, owT, ob):
    inv = float(1.0 / np.sqrt(HD))

    def body(q_ref, k_ref, v_ref, ow_ref, ob_ref, m_ref):
        p = q_ref[...] * k_ref[...]
        s = [jnp.sum(p[:, HD * h:HD * (h + 1)], axis=1, keepdims=True) * inv
             for h in range(NH)]
        mx = jnp.maximum(jnp.maximum(s[0], s[1]), jnp.maximum(s[2], s[3]))
        e = [jnp.exp(sh - mx) for sh in s]
        z = e[0] + e[1] + e[2] + e[3]
        w = jnp.concatenate(
            [jnp.broadcast_to(eh / z, (_EB, HD)) for eh in e], axis=1)
        attn = w * v_ref[...]
        out = jnp.dot(attn, ow_ref[...], preferred_element_type=F32) + ob_ref[...]
        m_ref[...] = _silu(out)
    return pl.pallas_call(
        body, grid=(E // _EB,),
        in_specs=[_espec(), _espec(), _espec(), _wspec(D, D), _wspec(1, D)],
        out_specs=_espec(),
        out_shape=jax.ShapeDtypeStruct((E, D), F32),
    )(gq, gk, gv, owT, ob)


def kernel(H_rna, C, edge_attr_rna, im_w1, im_b1, im_w2, im_b2, im_uw,
           im_ub, mc_g1w, mc_g1b, mc_g2w, mc_g2b, mc_mw, mc_mb, mc_uw,
           mc_ub, cm_qw, cm_qb, cm_kw, cm_kb, cm_vw, cm_vb, cm_ow, cm_ob,
           cm_g1w, cm_g1b, cm_g2w, cm_g2b, cc_w1, cc_b1, cc_w2, cc_b2,
           cc_uw, cc_ub, edge_index_rna, edge_index_belong,
           edge_index_cell):
    src, dst = edge_index_rna[0], edge_index_rna[1]
    bsrc, bdst = edge_index_belong[0], edge_index_belong[1]
    csrc, cdst = edge_index_cell[0], edge_index_cell[1]
    row = lambda b: b.reshape(1, -1)

    A, B = _tc_nodes1(H_rna, im_w1[:, :D].T, im_w1[:, D:2 * D].T,
                      row(im_b1))
    G1 = _g2a(dst, src, A, B)
    M1 = _tc_edge_mlp(G1, edge_attr_rna, im_w1[:, 2 * D:].T, im_w2.T,
                      row(im_b2))
    P1 = _scat(dst, M1)
    H1 = _tc_update(P1, H_rna, im_uw.T, row(im_ub))

    Xp, Cp, A16 = _tc_nodes2(H1, C, mc_mw[:, :D].T, mc_mw[:, D:].T,
                             row(mc_mb), mc_g1w.T, row(mc_g1b), mc_g2w,
                             mc_g2b.reshape(1, 1))
    P2 = _s2(bsrc, bdst, Xp, Cp)
    C1 = _tc_update_gated(P2, C, mc_uw.T, row(mc_ub), A16)

    Qn, Kn, Vn, G16 = _tc_nodes3(H1, C1, cm_qw.T, row(cm_qb), cm_kw.T,
                                 row(cm_kb), cm_vw.T, row(cm_vb),
                                 cm_g1w.T, row(cm_g1b), cm_g2w,
                                 cm_g2b.reshape(1, 1))
    GQ, GK, GV = _g3(bsrc, bdst, Qn, Kn, Vn)
    M3 = _tc_attn(GQ, GK, GV, cm_ow.T, row(cm_ob))
    P3 = _scat(bsrc, M3)
    Hout = _tc_add3_gated(P3, H1, H_rna, G16)

    A2, B2 = _tc_nodes1(C1, cc_w1[:, :D].T, cc_w1[:, D:].T, row(cc_b1))
    G4 = _g2a(cdst, csrc, A2, B2)
    M4 = _tc_edge_mlp(G4, None, None, cc_w2.T, row(cc_b2))
    P4 = _scat(cdst, M4)
    Cout = _tc_update_skip(P4, C1, cc_uw.T, row(cc_ub), C)

    return (Hout, Cout)


def _tc_update_skip(parts, x, uwT, ub, skip):
    def body(p0_ref, p1_ref, x_ref, uw_ref, ub_ref, s_ref, o_ref):
        s = p0_ref[0] + p1_ref[0]
        o_ref[...] = (x_ref[...]
                      + jnp.dot(s, uw_ref[...], preferred_element_type=F32)
                      + ub_ref[...] + s_ref[...])
    return pl.pallas_call(
        body, grid=(N // _RB,),
        in_specs=[_pspec(0), _pspec(1), _nspec(), _wspec(D, D),
                  _wspec(1, D), _nspec()],
        out_specs=_nspec(),
        out_shape=jax.ShapeDtypeStruct((N, D), F32),
    )(parts, parts, x, uwT, ub, skip)


# speedup vs baseline: 3.2223x; 1.0154x over previous
import functools
import numpy as np
import jax
import jax.numpy as jnp
from jax import lax
from jax.experimental import pallas as pl
from jax.experimental.pallas import tpu as pltpu
from jax.experimental.pallas import tpu_sc as plsc

N = 10000
D = 128
E = 320000
NH, HD = 4, 32
NC, NS = 2, 16
NW = NC * NS
RW = E // NW
CH = 80
NCHK = RW // CH
PN = 10240
TR = PN // NS
F32 = jnp.float32

_MESH = plsc.VectorSubcoreMesh(
    core_axis_name="c", subcore_axis_name="s", num_cores=NC, num_subcores=NS)


def _sig(x):
    return 1.0 / (1.0 + jnp.exp(-x))


def _silu(x):
    return x * _sig(x)


def _ids():
    cid = lax.axis_index("c")
    sid = lax.axis_index("s")
    return cid, sid, sid * NC + cid


def _zero_accum(zb, acc, sid):
    def zrow(r, _):
        for q in range(D // 16):
            zb[r, pl.ds(q * 16, 16)] = jnp.zeros((16,), F32)
        return 0
    lax.fori_loop(0, CH, zrow, 0)

    def zcp(k, _):
        pltpu.sync_copy(zb, acc.at[pl.ds(sid * TR + k * CH, CH)])
        return 0
    lax.fori_loop(0, TR // CH, zcp, 0)


def _dump_accum(acc, outh, cid, sid):
    def dcp(k, _):
        off = sid * TR + k * CH
        pltpu.sync_copy(acc.at[pl.ds(off, CH)],
                        outh.at[cid, pl.ds(off, CH)])
        return 0
    lax.fori_loop(0, TR // CH, dcp, 0)


def _pipe(prime, work):
    prime(0, 0)

    def pair(g, _):
        for b in (0, 1):
            j = 2 * g + b
            prime(j + 1, b ^ 1)
            work(j, b)
        return 0
    lax.fori_loop(0, (NCHK - 1) // 2, pair, 0)
    work(NCHK - 1, (NCHK - 1) % 2)


def _g2a_body(i1h, i2h, t1h, t2h, outh,
              i1v0, i1v1, i2v0, i2v1, b10, b11, b20, b21, sa0, sa1, sb0, sb1):
    _, _, wid = _ids()
    i1v, i2v = (i1v0, i1v1), (i2v0, i2v1)
    b1v, b2v = (b10, b11), (b20, b21)
    sa, sb = (sa0, sa1), (sb0, sb1)

    def prime(j, b):
        base = wid * RW + j * CH
        pltpu.sync_copy(i1h.at[pl.ds(base, CH)], i1v[b])
        pltpu.sync_copy(i2h.at[pl.ds(base, CH)], i2v[b])
        pltpu.async_copy(t1h.at[i1v[b]], b1v[b], sa[b])
        pltpu.async_copy(t2h.at[i2v[b]], b2v[b], sb[b])

    def work(j, b):
        base = wid * RW + j * CH
        pltpu.make_async_copy(t1h.at[i1v[b]], b1v[b], sa[b]).wait()
        pltpu.make_async_copy(t2h.at[i2v[b]], b2v[b], sb[b]).wait()

        @plsc.parallel_loop(0, CH, 1, unroll=4)
        def row(r):
            for q in range(D // 16):
                sl = pl.ds(q * 16, 16)
                b1v[b][r, sl] = b1v[b][r, sl] + b2v[b][r, sl]
        pltpu.sync_copy(b1v[b], outh.at[pl.ds(base, CH)])

    _pipe(prime, work)


_g2a = pl.kernel(
    _g2a_body,
    out_type=jax.ShapeDtypeStruct((E, D), F32),
    mesh=_MESH,
    scratch_types=[
        pltpu.VMEM((CH,), jnp.int32), pltpu.VMEM((CH,), jnp.int32),
        pltpu.VMEM((CH,), jnp.int32), pltpu.VMEM((CH,), jnp.int32),
        pltpu.VMEM((CH, D), F32), pltpu.VMEM((CH, D), F32),
        pltpu.VMEM((CH, D), F32), pltpu.VMEM((CH, D), F32),
        pltpu.SemaphoreType.DMA, pltpu.SemaphoreType.DMA,
        pltpu.SemaphoreType.DMA, pltpu.SemaphoreType.DMA,
    ],
)


def _g3_body(ibh, idh, qth, kth, vth, oqh, okh, ovh,
             ib0, ib1, id0, id1, bq0, bq1, bk0, bk1, bv0, bv1,
             sq0, sq1, sk0, sk1, sv0, sv1):
    _, _, wid = _ids()
    ibv, idv = (ib0, ib1), (id0, id1)
    bq, bk, bv = (bq0, bq1), (bk0, bk1), (bv0, bv1)
    sq, sk, sv = (sq0, sq1), (sk0, sk1), (sv0, sv1)

    def prime(j, b):
        base = wid * RW + j * CH
        pltpu.sync_copy(ibh.at[pl.ds(base, CH)], ibv[b])
        pltpu.sync_copy(idh.at[pl.ds(base, CH)], idv[b])
        pltpu.async_copy(qth.at[ibv[b]], bq[b], sq[b])
        pltpu.async_copy(kth.at[idv[b]], bk[b], sk[b])
        pltpu.async_copy(vth.at[idv[b]], bv[b], sv[b])

    def work(j, b):
        base = wid * RW + j * CH
        pltpu.make_async_copy(qth.at[ibv[b]], bq[b], sq[b]).wait()
        pltpu.make_async_copy(kth.at[idv[b]], bk[b], sk[b]).wait()
        pltpu.make_async_copy(vth.at[idv[b]], bv[b], sv[b]).wait()
        pltpu.sync_copy(bq[b], oqh.at[pl.ds(base, CH)])
        pltpu.sync_copy(bk[b], okh.at[pl.ds(base, CH)])
        pltpu.sync_copy(bv[b], ovh.at[pl.ds(base, CH)])

    _pipe(prime, work)


_g3 = pl.kernel(
    _g3_body,
    out_type=(jax.ShapeDtypeStruct((E, D), F32),
              jax.ShapeDtypeStruct((E, D), F32),
              jax.ShapeDtypeStruct((E, D), F32)),
    mesh=_MESH,
    scratch_types=(
        [pltpu.VMEM((CH,), jnp.int32)] * 4
        + [pltpu.VMEM((CH, D), F32)] * 6
        + [pltpu.SemaphoreType.DMA] * 6
    ),
)


def _make_scat():
    def body(idxh, mh, outh, iv0, iv1, bm0, bm1, s0, s1, acc):
        cid, sid, wid = _ids()
        iv, bm, sm = (iv0, iv1), (bm0, bm1), (s0, s1)
        _zero_accum(bm0, acc, sid)
        plsc.subcore_barrier()

        def prime(j, b):
            base = wid * RW + j * CH
            pltpu.sync_copy(idxh.at[pl.ds(base, CH)], iv[b])
            pltpu.async_copy(mh.at[pl.ds(base, CH)], bm[b], sm[b])

        def work(j, b):
            base = wid * RW + j * CH
            pltpu.make_async_copy(mh.at[pl.ds(base, CH)], bm[b], sm[b]).wait()
            pltpu.sync_copy(bm[b], acc.at[iv[b]], add=True)

        _pipe(prime, work)
        plsc.subcore_barrier()
        _dump_accum(acc, outh, cid, sid)

    return pl.kernel(
        body,
        out_type=jax.ShapeDtypeStruct((2, PN, D), F32),
        mesh=_MESH,
        scratch_types=[
            pltpu.VMEM((CH,), jnp.int32), pltpu.VMEM((CH,), jnp.int32),
            pltpu.VMEM((CH, D), F32), pltpu.VMEM((CH, D), F32),
            pltpu.SemaphoreType.DMA, pltpu.SemaphoreType.DMA,
            pltpu.VMEM_SHARED((PN, D), F32),
        ],
    )


_scat = _make_scat()


def _make_s2():
    def body(bsh, bdh, xph, cph, outh,
             ib0, ib1, id0, id1, bx0, bx1, bc0, bc1, sa0, sa1, sb0, sb1,
             acc):
        cid, sid, wid = _ids()
        ibv, idv = (ib0, ib1), (id0, id1)
        bx, bc = (bx0, bx1), (bc0, bc1)
        sa, sb = (sa0, sa1), (sb0, sb1)
        _zero_accum(bx0, acc, sid)
        plsc.subcore_barrier()

        def prime(j, b):
            base = wid * RW + j * CH
            pltpu.sync_copy(bsh.at[pl.ds(base, CH)], ibv[b])
            pltpu.sync_copy(bdh.at[pl.ds(base, CH)], idv[b])
            pltpu.async_copy(xph.at[ibv[b]], bx[b], sa[b])
            pltpu.async_copy(cph.at[idv[b]], bc[b], sb[b])

        def work(j, b):
            pltpu.make_async_copy(xph.at[ibv[b]], bx[b], sa[b]).wait()
            pltpu.make_async_copy(cph.at[idv[b]], bc[b], sb[b]).wait()

            @plsc.parallel_loop(0, CH, 1, unroll=2)
            def row(r):
                for q in range(D // 16):
                    sl = pl.ds(q * 16, 16)
                    x = bx[b][r, sl] + bc[b][r, sl]
                    x = x * (1.0 / (1.0 + jnp.exp(-x)))
                    bx[b][r, sl] = x * (1.0 / (1.0 + jnp.exp(-x)))
            pltpu.sync_copy(bx[b], acc.at[idv[b]], add=True)

        _pipe(prime, work)
        plsc.subcore_barrier()
        _dump_accum(acc, outh, cid, sid)

    return pl.kernel(
        body,
        out_type=jax.ShapeDtypeStruct((2, PN, D), F32),
        mesh=_MESH,
        scratch_types=(
            [pltpu.VMEM((CH,), jnp.int32)] * 4
            + [pltpu.VMEM((CH, D), F32)] * 4
            + [pltpu.SemaphoreType.DMA] * 4
            + [pltpu.VMEM_SHARED((PN, D), F32)]
        ),
    )


_s2 = _make_s2()


_RB = 1000
_EB = 2000


def _nspec(cols=D):
    return pl.BlockSpec((_RB, cols), lambda i: (i, 0))


def _wspec(r, c):
    return pl.BlockSpec((r, c), lambda i: (0, 0))


def _espec(cols=D):
    return pl.BlockSpec((_EB, cols), lambda i: (i, 0))


def _pspec(core):
    return pl.BlockSpec((1, _RB, D), lambda i, c=core: (c, i, 0))


def _tc_nodes1(h, w1dT, w1sT, b1):
    def body(h_ref, wd_ref, ws_ref, b1_ref, a_ref, b_ref):
        hh = h_ref[...]
        a_ref[...] = jnp.dot(hh, wd_ref[...], preferred_element_type=F32)
        b_ref[...] = jnp.dot(hh, ws_ref[...], preferred_element_type=F32) + b1_ref[...]
    return pl.pallas_call(
        body, grid=(N // _RB,),
        in_specs=[_nspec(), _wspec(D, D), _wspec(D, D), _wspec(1, D)],
        out_specs=[_nspec(), _nspec()],
        out_shape=[jax.ShapeDtypeStruct((N, D), F32)] * 2,
    )(h, w1dT, w1sT, b1)


def _tc_edge_mlp(g, ea, w1eT, w2T, b2):
    if ea is None:
        def body(g_ref, w2_ref, b2_ref, m_ref):
            m = _silu(g_ref[...])
            m = _silu(jnp.dot(m, w2_ref[...], preferred_element_type=F32)
                      + b2_ref[...])
            m_ref[...] = _silu(m)
        return pl.pallas_call(
            body, grid=(E // _EB,),
            in_specs=[_espec(), _wspec(D, D), _wspec(1, D)],
            out_specs=_espec(),
            out_shape=jax.ShapeDtypeStruct((E, D), F32),
        )(g, w2T, b2)

    def body(g_ref, ea_ref, w1e_ref, w2_ref, b2_ref, m_ref):
        pre = g_ref[...] + jnp.dot(ea_ref[...], w1e_ref[...],
                                   preferred_element_type=F32)
        m = _silu(pre)
        m = _silu(jnp.dot(m, w2_ref[...], preferred_element_type=F32)
                  + b2_ref[...])
        m_ref[...] = _silu(m)
    return pl.pallas_call(
        body, grid=(E // _EB,),
        in_specs=[_espec(), _espec(16), _wspec(16, D), _wspec(D, D),
                  _wspec(1, D)],
        out_specs=_espec(),
        out_shape=jax.ShapeDtypeStruct((E, D), F32),
    )(g, ea, w1eT, w2T, b2)


def _tc_update(parts, x, uwT, ub):
    def body(p0_ref, p1_ref, x_ref, uw_ref, ub_ref, o_ref):
        s = p0_ref[0] + p1_ref[0]
        o_ref[...] = (x_ref[...]
                      + jnp.dot(s, uw_ref[...], preferred_element_type=F32)
                      + ub_ref[...])
    return pl.pallas_call(
        body, grid=(N // _RB,),
        in_specs=[_pspec(0), _pspec(1),
                  _nspec(), _wspec(D, D), _wspec(1, D)],
        out_specs=_nspec(),
        out_shape=jax.ShapeDtypeStruct((N, D), F32),
    )(parts, parts, x, uwT, ub)


def _tc_nodes2(h1, c, mwxT, mwcT, mb, g1wT, g1b, g2w, g2b):
    def body(h1_ref, c_ref, wx_ref, wc_ref, mb_ref, g1_ref, g1b_ref,
             g2_ref, g2b_ref, xp_ref, cp_ref, a16_ref):
        cc = c_ref[...]
        xp_ref[...] = jnp.dot(h1_ref[...], wx_ref[...],
                              preferred_element_type=F32)
        cp_ref[...] = jnp.dot(cc, wc_ref[...],
                              preferred_element_type=F32) + mb_ref[...]
        t = _silu(jnp.dot(cc, g1_ref[...], preferred_element_type=F32)
                  + g1b_ref[...])
        a = _sig(jnp.sum(t * g2_ref[...], axis=1, keepdims=True)
                 + g2b_ref[...])
        a16_ref[...] = jnp.broadcast_to(a, (_RB, 16))
    return pl.pallas_call(
        body, grid=(N // _RB,),
        in_specs=[_nspec(), _nspec(), _wspec(D, D), _wspec(D, D),
                  _wspec(1, D), _wspec(D, D), _wspec(1, D), _wspec(1, D),
                  _wspec(1, 1)],
        out_specs=[_nspec(), _nspec(), _nspec(16)],
        out_shape=[jax.ShapeDtypeStruct((N, D), F32),
                   jax.ShapeDtypeStruct((N, D), F32),
                   jax.ShapeDtypeStruct((N, 16), F32)],
    )(h1, c, mwxT, mwcT, mb, g1wT, g1b, g2w, g2b)


def _tc_update_gated(parts, x, uwT, ub, a16):
    def body(p0_ref, p1_ref, x_ref, uw_ref, ub_ref, a_ref, o_ref):
        s = (p0_ref[0] + p1_ref[0]) * a_ref[:, 0:1]
        o_ref[...] = (x_ref[...]
                      + jnp.dot(s, uw_ref[...], preferred_element_type=F32)
                      + ub_ref[...])
    return pl.pallas_call(
        body, grid=(N // _RB,),
        in_specs=[_pspec(0), _pspec(1), _nspec(), _wspec(D, D),
                  _wspec(1, D), _nspec(16)],
        out_specs=_nspec(),
        out_shape=jax.ShapeDtypeStruct((N, D), F32),
    )(parts, parts, x, uwT, ub, a16)


def _tc_add3_gated(parts, x, skip, g16):
    def body(p0_ref, p1_ref, x_ref, s_ref, g_ref, o_ref):
        o_ref[...] = ((p0_ref[0] + p1_ref[0]) * g_ref[:, 0:1]
                      + x_ref[...] + s_ref[...])
    return pl.pallas_call(
        body, grid=(N // _RB,),
        in_specs=[_pspec(0), _pspec(1), _nspec(), _nspec(), _nspec(16)],
        out_specs=_nspec(),
        out_shape=jax.ShapeDtypeStruct((N, D), F32),
    )(parts, parts, x, skip, g16)


def _tc_nodes3(h1, c1, qwT, qb, kwT, kb, vwT, vb, g1wT, g1b, g2w, g2b):
    def body(h1_ref, c1_ref, qw_ref, qb_ref, kw_ref, kb_ref, vw_ref,
             vb_ref, g1_ref, g1b_ref, g2_ref, g2b_ref,
             q_ref, k_ref, v_ref, g16_ref):
        hh = h1_ref[...]
        cc = c1_ref[...]
        q_ref[...] = jnp.dot(hh, qw_ref[...], preferred_element_type=F32) + qb_ref[...]
        k_ref[...] = jnp.dot(cc, kw_ref[...], preferred_element_type=F32) + kb_ref[...]
        v_ref[...] = jnp.dot(cc, vw_ref[...], preferred_element_type=F32) + vb_ref[...]
        t = _silu(jnp.dot(hh, g1_ref[...], preferred_element_type=F32)
                  + g1b_ref[...])
        g = _sig(jnp.sum(t * g2_ref[...], axis=1, keepdims=True)
                 + g2b_ref[...])
        g16_ref[...] = jnp.broadcast_to(g, (_RB, 16))
    return pl.pallas_call(
        body, grid=(N // _RB,),
        in_specs=[_nspec(), _nspec(), _wspec(D, D), _wspec(1, D),
                  _wspec(D, D), _wspec(1, D), _wspec(D, D), _wspec(1, D),
                  _wspec(D, D), _wspec(1, D), _wspec(1, D), _wspec(1, 1)],
        out_specs=[_nspec(), _nspec(), _nspec(), _nspec(16)],
        out_shape=[jax.ShapeDtypeStruct((N, D), F32),
                   jax.ShapeDtypeStruct((N, D), F32),
                   jax.ShapeDtypeStruct((N, D), F32),
                   jax.ShapeDtypeStruct((N, 16), F32)],
    )(h1, c1, qwT, qb, kwT, kb, vwT, vb, g1wT, g1b, g2w, g2b)


def _tc_attn(gq, gk, gv, owT, ob):
    inv = float(1.0 / np.sqrt(HD))

    def body(q_ref, k_ref, v_ref, ow_ref, ob_ref, m_ref):
        p = q_ref[...] * k_ref[...]
        s = [jnp.sum(p[:, HD * h:HD * (h + 1)], axis=1, keepdims=True) * inv
             for h in range(NH)]
        mx = jnp.maximum(jnp.maximum(s[0], s[1]), jnp.maximum(s[2], s[3]))
        e = [jnp.exp(sh - mx) for sh in s]
        z = e[0] + e[1] + e[2] + e[3]
        w = jnp.concatenate(
            [jnp.broadcast_to(eh / z, (_EB, HD)) for eh in e], axis=1)
        attn = w * v_ref[...]
        out = jnp.dot(attn, ow_ref[...], preferred_element_type=F32) + ob_ref[...]
        m_ref[...] = _silu(out)
    return pl.pallas_call(
        body, grid=(E // _EB,),
        in_specs=[_espec(), _espec(), _espec(), _wspec(D, D), _wspec(1, D)],
        out_specs=_espec(),
        out_shape=jax.ShapeDtypeStruct((E, D), F32),
    )(gq, gk, gv, owT, ob)


def kernel(H_rna, C, edge_attr_rna, im_w1, im_b1, im_w2, im_b2, im_uw,
           im_ub, mc_g1w, mc_g1b, mc_g2w, mc_g2b, mc_mw, mc_mb, mc_uw,
           mc_ub, cm_qw, cm_qb, cm_kw, cm_kb, cm_vw, cm_vb, cm_ow, cm_ob,
           cm_g1w, cm_g1b, cm_g2w, cm_g2b, cc_w1, cc_b1, cc_w2, cc_b2,
           cc_uw, cc_ub, edge_index_rna, edge_index_belong,
           edge_index_cell):
    src, dst = edge_index_rna[0], edge_index_rna[1]
    bsrc, bdst = edge_index_belong[0], edge_index_belong[1]
    csrc, cdst = edge_index_cell[0], edge_index_cell[1]
    row = lambda b: b.reshape(1, -1)

    A, B = _tc_nodes1(H_rna, im_w1[:, :D].T, im_w1[:, D:2 * D].T,
                      row(im_b1))
    G1 = _g2a(dst, src, A, B)
    M1 = _tc_edge_mlp(G1, edge_attr_rna, im_w1[:, 2 * D:].T, im_w2.T,
                      row(im_b2))
    P1 = _scat(dst, M1)
    H1 = _tc_update(P1, H_rna, im_uw.T, row(im_ub))

    Xp, Cp, A16 = _tc_nodes2(H1, C, mc_mw[:, :D].T, mc_mw[:, D:].T,
                             row(mc_mb), mc_g1w.T, row(mc_g1b), mc_g2w,
                             mc_g2b.reshape(1, 1))
    P2 = _s2(bsrc, bdst, Xp, Cp)
    C1 = _tc_update_gated(P2, C, mc_uw.T, row(mc_ub), A16)

    Qn, Kn, Vn, G16 = _tc_nodes3(H1, C1, cm_qw.T, row(cm_qb), cm_kw.T,
                                 row(cm_kb), cm_vw.T, row(cm_vb),
                                 cm_g1w.T, row(cm_g1b), cm_g2w,
                                 cm_g2b.reshape(1, 1))
    GQ, GK, GV = _g3(bsrc, bdst, Qn, Kn, Vn)
    M3 = _tc_attn(GQ, GK, GV, cm_ow.T, row(cm_ob))
    P3 = _scat(bsrc, M3)
    Hout = _tc_add3_gated(P3, H1, H_rna, G16)

    A2, B2 = _tc_nodes1(C1, cc_w1[:, :D].T, cc_w1[:, D:].T, row(cc_b1))
    G4 = _g2a(cdst, csrc, A2, B2)
    M4 = _tc_edge_mlp(G4, None, None, cc_w2.T, row(cc_b2))
    P4 = _scat(cdst, M4)
    Cout = _tc_update_skip(P4, C1, cc_uw.T, row(cc_ub), C)

    return (Hout, Cout)


def _tc_update_skip(parts, x, uwT, ub, skip):
    def body(p0_ref, p1_ref, x_ref, uw_ref, ub_ref, s_ref, o_ref):
        s = p0_ref[0] + p1_ref[0]
        o_ref[...] = (x_ref[...]
                      + jnp.dot(s, uw_ref[...], preferred_element_type=F32)
                      + ub_ref[...] + s_ref[...])
    return pl.pallas_call(
        body, grid=(N // _RB,),
        in_specs=[_pspec(0), _pspec(1), _nspec(), _wspec(D, D),
                  _wspec(1, D), _nspec()],
        out_specs=_nspec(),
        out_shape=jax.ShapeDtypeStruct((N, D), F32),
    )(parts, parts, x, uwT, ub, skip)


# speedup vs baseline: 3.4975x; 1.0854x over previous
import functools
import numpy as np
import jax
import jax.numpy as jnp
from jax import lax
from jax.experimental import pallas as pl
from jax.experimental.pallas import tpu as pltpu
from jax.experimental.pallas import tpu_sc as plsc

N = 10000
D = 128
E = 320000
NH, HD = 4, 32
NC, NS = 2, 16
NW = NC * NS
RW = E // NW
CH = 80
NCHK = RW // CH
PN = 10240
TR = PN // NS
F32 = jnp.float32

_MESH = plsc.VectorSubcoreMesh(
    core_axis_name="c", subcore_axis_name="s", num_cores=NC, num_subcores=NS)


def _sig(x):
    return 1.0 / (1.0 + jnp.exp(-x))


def _silu(x):
    return x * _sig(x)


def _ids():
    cid = lax.axis_index("c")
    sid = lax.axis_index("s")
    return cid, sid, sid * NC + cid


def _zero_accum(zb, acc, sid):
    def zrow(r, _):
        for q in range(D // 16):
            zb[r, pl.ds(q * 16, 16)] = jnp.zeros((16,), F32)
        return 0
    lax.fori_loop(0, CH, zrow, 0)

    def zcp(k, _):
        pltpu.sync_copy(zb, acc.at[pl.ds(sid * TR + k * CH, CH)])
        return 0
    lax.fori_loop(0, TR // CH, zcp, 0)


def _dump_accum(acc, outh, cid, sid):
    def dcp(k, _):
        off = sid * TR + k * CH
        pltpu.sync_copy(acc.at[pl.ds(off, CH)],
                        outh.at[cid, pl.ds(off, CH)])
        return 0
    lax.fori_loop(0, TR // CH, dcp, 0)


def _pipe(nchk, prime, work):
    prime(0, 0)

    def pair(g, _):
        for b in (0, 1):
            j = 2 * g + b

            @pl.when(j + 1 < nchk)
            def _():
                prime(j + 1, b ^ 1)
            work(j, b)
        return 0
    lax.fori_loop(0, nchk // 2, pair, 0)
    if nchk % 2 == 1:
        work(nchk - 1, 0)


def _make_g2a(rw):
    nchk = rw // CH

    def body(i1h, i2h, t1h, t2h, outh,
             i1v0, i1v1, i2v0, i2v1, b10, b11, b20, b21,
             sa0, sa1, sb0, sb1):
        _, _, wid = _ids()
        i1v, i2v = (i1v0, i1v1), (i2v0, i2v1)
        b1v, b2v = (b10, b11), (b20, b21)
        sa, sb = (sa0, sa1), (sb0, sb1)

        def prime(j, b):
            base = wid * rw + j * CH
            pltpu.sync_copy(i1h.at[pl.ds(base, CH)], i1v[b])
            pltpu.sync_copy(i2h.at[pl.ds(base, CH)], i2v[b])
            pltpu.async_copy(t1h.at[i1v[b]], b1v[b], sa[b])
            pltpu.async_copy(t2h.at[i2v[b]], b2v[b], sb[b])

        def work(j, b):
            base = wid * rw + j * CH
            pltpu.make_async_copy(t1h.at[i1v[b]], b1v[b], sa[b]).wait()
            pltpu.make_async_copy(t2h.at[i2v[b]], b2v[b], sb[b]).wait()

            @plsc.parallel_loop(0, CH, 1, unroll=4)
            def row(r):
                for q in range(D // 16):
                    sl = pl.ds(q * 16, 16)
                    b1v[b][r, sl] = b1v[b][r, sl] + b2v[b][r, sl]
            pltpu.sync_copy(b1v[b], outh.at[pl.ds(base, CH)])

        _pipe(nchk, prime, work)

    return pl.kernel(
        body,
        out_type=jax.ShapeDtypeStruct((rw * NW, D), F32),
        mesh=_MESH,
        scratch_types=(
            [pltpu.VMEM((CH,), jnp.int32)] * 4
            + [pltpu.VMEM((CH, D), F32)] * 4
            + [pltpu.SemaphoreType.DMA] * 4
        ),
    )


def _make_g3(rw):
    nchk = rw // CH

    def body(ibh, idh, qth, kth, vth, oqh, okh, ovh,
             ib0, ib1, id0, id1, bq0, bq1, bk0, bk1, bv0, bv1,
             sq0, sq1, sk0, sk1, sv0, sv1):
        _, _, wid = _ids()
        ibv, idv = (ib0, ib1), (id0, id1)
        bq, bk, bv = (bq0, bq1), (bk0, bk1), (bv0, bv1)
        sq, sk, sv = (sq0, sq1), (sk0, sk1), (sv0, sv1)

        def prime(j, b):
            base = wid * rw + j * CH
            pltpu.sync_copy(ibh.at[pl.ds(base, CH)], ibv[b])
            pltpu.sync_copy(idh.at[pl.ds(base, CH)], idv[b])
            pltpu.async_copy(qth.at[ibv[b]], bq[b], sq[b])
            pltpu.async_copy(kth.at[idv[b]], bk[b], sk[b])
            pltpu.async_copy(vth.at[idv[b]], bv[b], sv[b])

        def work(j, b):
            base = wid * rw + j * CH
            pltpu.make_async_copy(qth.at[ibv[b]], bq[b], sq[b]).wait()
            pltpu.make_async_copy(kth.at[idv[b]], bk[b], sk[b]).wait()
            pltpu.make_async_copy(vth.at[idv[b]], bv[b], sv[b]).wait()
            pltpu.sync_copy(bq[b], oqh.at[pl.ds(base, CH)])
            pltpu.sync_copy(bk[b], okh.at[pl.ds(base, CH)])
            pltpu.sync_copy(bv[b], ovh.at[pl.ds(base, CH)])

        _pipe(nchk, prime, work)

    return pl.kernel(
        body,
        out_type=(jax.ShapeDtypeStruct((rw * NW, D), F32),
                  jax.ShapeDtypeStruct((rw * NW, D), F32),
                  jax.ShapeDtypeStruct((rw * NW, D), F32)),
        mesh=_MESH,
        scratch_types=(
            [pltpu.VMEM((CH,), jnp.int32)] * 4
            + [pltpu.VMEM((CH, D), F32)] * 6
            + [pltpu.SemaphoreType.DMA] * 6
        ),
    )


def _make_scat(rw):
    nchk = rw // CH

    def body(idxh, mh, outh, iv0, iv1, bm0, bm1, s0, s1, acc):
        cid, sid, wid = _ids()
        iv, bm, sm = (iv0, iv1), (bm0, bm1), (s0, s1)
        _zero_accum(bm0, acc, sid)
        plsc.subcore_barrier()

        def prime(j, b):
            base = wid * rw + j * CH
            pltpu.sync_copy(idxh.at[pl.ds(base, CH)], iv[b])
            pltpu.async_copy(mh.at[pl.ds(base, CH)], bm[b], sm[b])

        def work(j, b):
            base = wid * rw + j * CH
            pltpu.make_async_copy(mh.at[pl.ds(base, CH)], bm[b], sm[b]).wait()
            pltpu.sync_copy(bm[b], acc.at[iv[b]], add=True)

        _pipe(nchk, prime, work)
        plsc.subcore_barrier()
        _dump_accum(acc, outh, cid, sid)

    return pl.kernel(
        body,
        out_type=jax.ShapeDtypeStruct((2, PN, D), F32),
        mesh=_MESH,
        scratch_types=[
            pltpu.VMEM((CH,), jnp.int32), pltpu.VMEM((CH,), jnp.int32),
            pltpu.VMEM((CH, D), F32), pltpu.VMEM((CH, D), F32),
            pltpu.SemaphoreType.DMA, pltpu.SemaphoreType.DMA,
            pltpu.VMEM_SHARED((PN, D), F32),
        ],
    )


EA_, EB_ = 192000, 128000
RWA, RWB = EA_ // NW, EB_ // NW
_g2a_a, _g2a_b = _make_g2a(RWA), _make_g2a(RWB)
_g3_a, _g3_b = _make_g3(RWA), _make_g3(RWB)
_scat_a, _scat_b = _make_scat(RWA), _make_scat(RWB)


def _make_s2():
    def body(bsh, bdh, xph, cph, outh,
             ib0, ib1, id0, id1, bx0, bx1, bc0, bc1, sa0, sa1, sb0, sb1,
             acc):
        cid, sid, wid = _ids()
        ibv, idv = (ib0, ib1), (id0, id1)
        bx, bc = (bx0, bx1), (bc0, bc1)
        sa, sb = (sa0, sa1), (sb0, sb1)
        _zero_accum(bx0, acc, sid)
        plsc.subcore_barrier()

        def prime(j, b):
            base = wid * RW + j * CH
            pltpu.sync_copy(bsh.at[pl.ds(base, CH)], ibv[b])
            pltpu.sync_copy(bdh.at[pl.ds(base, CH)], idv[b])
            pltpu.async_copy(xph.at[ibv[b]], bx[b], sa[b])
            pltpu.async_copy(cph.at[idv[b]], bc[b], sb[b])

        def work(j, b):
            pltpu.make_async_copy(xph.at[ibv[b]], bx[b], sa[b]).wait()
            pltpu.make_async_copy(cph.at[idv[b]], bc[b], sb[b]).wait()

            @plsc.parallel_loop(0, CH, 1, unroll=2)
            def row(r):
                for q in range(D // 16):
                    sl = pl.ds(q * 16, 16)
                    x = bx[b][r, sl] + bc[b][r, sl]
                    x = x * (1.0 / (1.0 + jnp.exp(-x)))
                    bx[b][r, sl] = x * (1.0 / (1.0 + jnp.exp(-x)))
            pltpu.sync_copy(bx[b], acc.at[idv[b]], add=True)

        _pipe(NCHK, prime, work)
        plsc.subcore_barrier()
        _dump_accum(acc, outh, cid, sid)

    return pl.kernel(
        body,
        out_type=jax.ShapeDtypeStruct((2, PN, D), F32),
        mesh=_MESH,
        scratch_types=(
            [pltpu.VMEM((CH,), jnp.int32)] * 4
            + [pltpu.VMEM((CH, D), F32)] * 4
            + [pltpu.SemaphoreType.DMA] * 4
            + [pltpu.VMEM_SHARED((PN, D), F32)]
        ),
    )


_s2 = _make_s2()


_RB = 1000
_EB = 2000


def _nspec(cols=D):
    return pl.BlockSpec((_RB, cols), lambda i: (i, 0))


def _wspec(r, c):
    return pl.BlockSpec((r, c), lambda i: (0, 0))


def _espec(cols=D):
    return pl.BlockSpec((_EB, cols), lambda i: (i, 0))


def _pspec(core):
    return pl.BlockSpec((1, _RB, D), lambda i, c=core: (c, i, 0))


def _tc_nodes1(h, w1dT, w1sT, b1):
    def body(h_ref, wd_ref, ws_ref, b1_ref, a_ref, b_ref):
        hh = h_ref[...]
        a_ref[...] = jnp.dot(hh, wd_ref[...], preferred_element_type=F32)
        b_ref[...] = jnp.dot(hh, ws_ref[...], preferred_element_type=F32) + b1_ref[...]
    return pl.pallas_call(
        body, grid=(N // _RB,),
        in_specs=[_nspec(), _wspec(D, D), _wspec(D, D), _wspec(1, D)],
        out_specs=[_nspec(), _nspec()],
        out_shape=[jax.ShapeDtypeStruct((N, D), F32)] * 2,
    )(h, w1dT, w1sT, b1)


def _tc_edge_mlp(g, ea, w1eT, w2T, b2):
    esz = g.shape[0]
    if ea is None:
        def body(g_ref, w2_ref, b2_ref, m_ref):
            m = _silu(g_ref[...])
            m = _silu(jnp.dot(m, w2_ref[...], preferred_element_type=F32)
                      + b2_ref[...])
            m_ref[...] = _silu(m)
        return pl.pallas_call(
            body, grid=(esz // _EB,),
            in_specs=[_espec(), _wspec(D, D), _wspec(1, D)],
            out_specs=_espec(),
            out_shape=jax.ShapeDtypeStruct((esz, D), F32),
        )(g, w2T, b2)

    def body(g_ref, ea_ref, w1e_ref, w2_ref, b2_ref, m_ref):
        pre = g_ref[...] + jnp.dot(ea_ref[...], w1e_ref[...],
                                   preferred_element_type=F32)
        m = _silu(pre)
        m = _silu(jnp.dot(m, w2_ref[...], preferred_element_type=F32)
                  + b2_ref[...])
        m_ref[...] = _silu(m)
    return pl.pallas_call(
        body, grid=(esz // _EB,),
        in_specs=[_espec(), _espec(16), _wspec(16, D), _wspec(D, D),
                  _wspec(1, D)],
        out_specs=_espec(),
        out_shape=jax.ShapeDtypeStruct((esz, D), F32),
    )(g, ea, w1eT, w2T, b2)


def _tc_update(pa, pb, x, uwT, ub):
    def body(p0_ref, p1_ref, p2_ref, p3_ref, x_ref, uw_ref, ub_ref, o_ref):
        s = p0_ref[0] + p1_ref[0] + p2_ref[0] + p3_ref[0]
        o_ref[...] = (x_ref[...]
                      + jnp.dot(s, uw_ref[...], preferred_element_type=F32)
                      + ub_ref[...])
    return pl.pallas_call(
        body, grid=(N // _RB,),
        in_specs=[_pspec(0), _pspec(1), _pspec(0), _pspec(1),
                  _nspec(), _wspec(D, D), _wspec(1, D)],
        out_specs=_nspec(),
        out_shape=jax.ShapeDtypeStruct((N, D), F32),
    )(pa, pa, pb, pb, x, uwT, ub)


def _tc_nodes2(h1, c, mwxT, mwcT, mb, g1wT, g1b, g2w, g2b):
    def body(h1_ref, c_ref, wx_ref, wc_ref, mb_ref, g1_ref, g1b_ref,
             g2_ref, g2b_ref, xp_ref, cp_ref, a16_ref):
        cc = c_ref[...]
        xp_ref[...] = jnp.dot(h1_ref[...], wx_ref[...],
                              preferred_element_type=F32)
        cp_ref[...] = jnp.dot(cc, wc_ref[...],
                              preferred_element_type=F32) + mb_ref[...]
        t = _silu(jnp.dot(cc, g1_ref[...], preferred_element_type=F32)
                  + g1b_ref[...])
        a = _sig(jnp.sum(t * g2_ref[...], axis=1, keepdims=True)
                 + g2b_ref[...])
        a16_ref[...] = jnp.broadcast_to(a, (_RB, 16))
    return pl.pallas_call(
        body, grid=(N // _RB,),
        in_specs=[_nspec(), _nspec(), _wspec(D, D), _wspec(D, D),
                  _wspec(1, D), _wspec(D, D), _wspec(1, D), _wspec(1, D),
                  _wspec(1, 1)],
        out_specs=[_nspec(), _nspec(), _nspec(16)],
        out_shape=[jax.ShapeDtypeStruct((N, D), F32),
                   jax.ShapeDtypeStruct((N, D), F32),
                   jax.ShapeDtypeStruct((N, 16), F32)],
    )(h1, c, mwxT, mwcT, mb, g1wT, g1b, g2w, g2b)


def _tc_update_gated(parts, x, uwT, ub, a16):
    def body(p0_ref, p1_ref, x_ref, uw_ref, ub_ref, a_ref, o_ref):
        s = (p0_ref[0] + p1_ref[0]) * a_ref[:, 0:1]
        o_ref[...] = (x_ref[...]
                      + jnp.dot(s, uw_ref[...], preferred_element_type=F32)
                      + ub_ref[...])
    return pl.pallas_call(
        body, grid=(N // _RB,),
        in_specs=[_pspec(0), _pspec(1), _nspec(), _wspec(D, D),
                  _wspec(1, D), _nspec(16)],
        out_specs=_nspec(),
        out_shape=jax.ShapeDtypeStruct((N, D), F32),
    )(parts, parts, x, uwT, ub, a16)


def _tc_add3_gated(pa, pb, x, skip, g16):
    def body(p0_ref, p1_ref, p2_ref, p3_ref, x_ref, s_ref, g_ref, o_ref):
        ps = p0_ref[0] + p1_ref[0] + p2_ref[0] + p3_ref[0]
        o_ref[...] = ps * g_ref[:, 0:1] + x_ref[...] + s_ref[...]
    return pl.pallas_call(
        body, grid=(N // _RB,),
        in_specs=[_pspec(0), _pspec(1), _pspec(0), _pspec(1),
                  _nspec(), _nspec(), _nspec(16)],
        out_specs=_nspec(),
        out_shape=jax.ShapeDtypeStruct((N, D), F32),
    )(pa, pa, pb, pb, x, skip, g16)


def _tc_nodes3a(h1, qwT, qb, g1wT, g1b, g2w, g2b):
    def body(h1_ref, qw_ref, qb_ref, g1_ref, g1b_ref, g2_ref, g2b_ref,
             q_ref, g16_ref):
        hh = h1_ref[...]
        q_ref[...] = jnp.dot(hh, qw_ref[...], preferred_element_type=F32) + qb_ref[...]
        t = _silu(jnp.dot(hh, g1_ref[...], preferred_element_type=F32)
                  + g1b_ref[...])
        g = _sig(jnp.sum(t * g2_ref[...], axis=1, keepdims=True)
                 + g2b_ref[...])
        g16_ref[...] = jnp.broadcast_to(g, (_RB, 16))
    return pl.pallas_call(
        body, grid=(N // _RB,),
        in_specs=[_nspec(), _wspec(D, D), _wspec(1, D),
                  _wspec(D, D), _wspec(1, D), _wspec(1, D), _wspec(1, 1)],
        out_specs=[_nspec(), _nspec(16)],
        out_shape=[jax.ShapeDtypeStruct((N, D), F32),
                   jax.ShapeDtypeStruct((N, 16), F32)],
    )(h1, qwT, qb, g1wT, g1b, g2w, g2b)


def _tc_nodes3b(c1, kwT, kb, vwT, vb):
    def body(c1_ref, kw_ref, kb_ref, vw_ref, vb_ref, k_ref, v_ref):
        cc = c1_ref[...]
        k_ref[...] = jnp.dot(cc, kw_ref[...], preferred_element_type=F32) + kb_ref[...]
        v_ref[...] = jnp.dot(cc, vw_ref[...], preferred_element_type=F32) + vb_ref[...]
    return pl.pallas_call(
        body, grid=(N // _RB,),
        in_specs=[_nspec(), _wspec(D, D), _wspec(1, D), _wspec(D, D),
                  _wspec(1, D)],
        out_specs=[_nspec(), _nspec()],
        out_shape=[jax.ShapeDtypeStruct((N, D), F32),
                   jax.ShapeDtypeStruct((N, D), F32)],
    )(c1, kwT, kb, vwT, vb)


def _tc_attn(gq, gk, gv, owT, ob):
    inv = float(1.0 / np.sqrt(HD))
    esz = gq.shape[0]

    def body(q_ref, k_ref, v_ref, ow_ref, ob_ref, m_ref):
        p = q_ref[...] * k_ref[...]
        s = [jnp.sum(p[:, HD * h:HD * (h + 1)], axis=1, keepdims=True) * inv
             for h in range(NH)]
        mx = jnp.maximum(jnp.maximum(s[0], s[1]), jnp.maximum(s[2], s[3]))
        e = [jnp.exp(sh - mx) for sh in s]
        z = e[0] + e[1] + e[2] + e[3]
        w = jnp.concatenate(
            [jnp.broadcast_to(eh / z, (_EB, HD)) for eh in e], axis=1)
        attn = w * v_ref[...]
        out = jnp.dot(attn, ow_ref[...], preferred_element_type=F32) + ob_ref[...]
        m_ref[...] = _silu(out)
    return pl.pallas_call(
        body, grid=(esz // _EB,),
        in_specs=[_espec(), _espec(), _espec(), _wspec(D, D), _wspec(1, D)],
        out_specs=_espec(),
        out_shape=jax.ShapeDtypeStruct((esz, D), F32),
    )(gq, gk, gv, owT, ob)


def kernel(H_rna, C, edge_attr_rna, im_w1, im_b1, im_w2, im_b2, im_uw,
           im_ub, mc_g1w, mc_g1b, mc_g2w, mc_g2b, mc_mw, mc_mb, mc_uw,
           mc_ub, cm_qw, cm_qb, cm_kw, cm_kb, cm_vw, cm_vb, cm_ow, cm_ob,
           cm_g1w, cm_g1b, cm_g2w, cm_g2b, cc_w1, cc_b1, cc_w2, cc_b2,
           cc_uw, cc_ub, edge_index_rna, edge_index_belong,
           edge_index_cell):
    src, dst = edge_index_rna[0], edge_index_rna[1]
    bsrc, bdst = edge_index_belong[0], edge_index_belong[1]
    csrc, cdst = edge_index_cell[0], edge_index_cell[1]
    row = lambda b: b.reshape(1, -1)
    half = lambda v: (v[:EA_], v[EA_:])

    src_a, src_b = half(src)
    dst_a, dst_b = half(dst)
    ea_a, ea_b = edge_attr_rna[:EA_], edge_attr_rna[EA_:]
    A, B = _tc_nodes1(H_rna, im_w1[:, :D].T, im_w1[:, D:2 * D].T,
                      row(im_b1))
    w1eT, w2T, b2 = im_w1[:, 2 * D:].T, im_w2.T, row(im_b2)
    G1a = _g2a_a(dst_a, src_a, A, B)
    M1a = _tc_edge_mlp(G1a, ea_a, w1eT, w2T, b2)
    G1b = _g2a_b(dst_b, src_b, A, B)
    M1b = _tc_edge_mlp(G1b, ea_b, w1eT, w2T, b2)
    P1a = _scat_a(dst_a, M1a)
    P1b = _scat_b(dst_b, M1b)
    H1 = _tc_update(P1a, P1b, H_rna, im_uw.T, row(im_ub))

    Xp, Cp, A16 = _tc_nodes2(H1, C, mc_mw[:, :D].T, mc_mw[:, D:].T,
                             row(mc_mb), mc_g1w.T, row(mc_g1b), mc_g2w,
                             mc_g2b.reshape(1, 1))
    P2 = _s2(bsrc, bdst, Xp, Cp)
    Qn, G16 = _tc_nodes3a(H1, cm_qw.T, row(cm_qb), cm_g1w.T, row(cm_g1b),
                          cm_g2w, cm_g2b.reshape(1, 1))
    C1 = _tc_update_gated(P2, C, mc_uw.T, row(mc_ub), A16)

    bsrc_a, bsrc_b = half(bsrc)
    bdst_a, bdst_b = half(bdst)
    Kn, Vn = _tc_nodes3b(C1, cm_kw.T, row(cm_kb), cm_vw.T, row(cm_vb))
    owT, ob = cm_ow.T, row(cm_ob)
    GQa, GKa, GVa = _g3_a(bsrc_a, bdst_a, Qn, Kn, Vn)
    M3a = _tc_attn(GQa, GKa, GVa, owT, ob)
    GQb, GKb, GVb = _g3_b(bsrc_b, bdst_b, Qn, Kn, Vn)
    M3b = _tc_attn(GQb, GKb, GVb, owT, ob)
    P3a = _scat_a(bsrc_a, M3a)
    P3b = _scat_b(bsrc_b, M3b)
    Hout = _tc_add3_gated(P3a, P3b, H1, H_rna, G16)

    csrc_a, csrc_b = half(csrc)
    cdst_a, cdst_b = half(cdst)
    A2, B2 = _tc_nodes1(C1, cc_w1[:, :D].T, cc_w1[:, D:].T, row(cc_b1))
    cw2T, cb2 = cc_w2.T, row(cc_b2)
    G4a = _g2a_a(cdst_a, csrc_a, A2, B2)
    M4a = _tc_edge_mlp(G4a, None, None, cw2T, cb2)
    G4b = _g2a_b(cdst_b, csrc_b, A2, B2)
    M4b = _tc_edge_mlp(G4b, None, None, cw2T, cb2)
    P4a = _scat_a(cdst_a, M4a)
    P4b = _scat_b(cdst_b, M4b)
    Cout = _tc_update_skip(P4a, P4b, C1, cc_uw.T, row(cc_ub), C)

    return (Hout, Cout)


def _tc_update_skip(pa, pb, x, uwT, ub, skip):
    def body(p0_ref, p1_ref, p2_ref, p3_ref, x_ref, uw_ref, ub_ref,
             s_ref, o_ref):
        s = p0_ref[0] + p1_ref[0] + p2_ref[0] + p3_ref[0]
        o_ref[...] = (x_ref[...]
                      + jnp.dot(s, uw_ref[...], preferred_element_type=F32)
                      + ub_ref[...] + s_ref[...])
    return pl.pallas_call(
        body, grid=(N // _RB,),
        in_specs=[_pspec(0), _pspec(1), _pspec(0), _pspec(1), _nspec(),
                  _wspec(D, D), _wspec(1, D), _nspec()],
        out_specs=_nspec(),
        out_shape=jax.ShapeDtypeStruct((N, D), F32),
    )(pa, pa, pb, pb, x, uwT, ub, skip)


# speedup vs baseline: 3.5047x; 1.0021x over previous
import functools
import numpy as np
import jax
import jax.numpy as jnp
from jax import lax
from jax.experimental import pallas as pl
from jax.experimental.pallas import tpu as pltpu
from jax.experimental.pallas import tpu_sc as plsc

N = 10000
D = 128
E = 320000
NH, HD = 4, 32
NC, NS = 2, 16
NW = NC * NS
RW = E // NW
CH = 80
NCHK = RW // CH
PN = 10240
TR = PN // NS
F32 = jnp.float32

_MESH = plsc.VectorSubcoreMesh(
    core_axis_name="c", subcore_axis_name="s", num_cores=NC, num_subcores=NS)


def _sig(x):
    return 1.0 / (1.0 + jnp.exp(-x))


def _silu(x):
    return x * _sig(x)


def _ids():
    cid = lax.axis_index("c")
    sid = lax.axis_index("s")
    return cid, sid, sid * NC + cid


def _zero_accum(zb, acc, sid):
    def zrow(r, _):
        for q in range(D // 16):
            zb[r, pl.ds(q * 16, 16)] = jnp.zeros((16,), F32)
        return 0
    lax.fori_loop(0, CH, zrow, 0)

    def zcp(k, _):
        pltpu.sync_copy(zb, acc.at[pl.ds(sid * TR + k * CH, CH)])
        return 0
    lax.fori_loop(0, TR // CH, zcp, 0)


def _dump_accum(acc, outh, cid, sid):
    def dcp(k, _):
        off = sid * TR + k * CH
        pltpu.sync_copy(acc.at[pl.ds(off, CH)],
                        outh.at[cid, pl.ds(off, CH)])
        return 0
    lax.fori_loop(0, TR // CH, dcp, 0)


def _pipe(nchk, prime, work):
    prime(0, 0)

    def pair(g, _):
        for b in (0, 1):
            j = 2 * g + b

            @pl.when(j + 1 < nchk)
            def _():
                prime(j + 1, b ^ 1)
            work(j, b)
        return 0
    lax.fori_loop(0, nchk // 2, pair, 0)
    if nchk % 2 == 1:
        work(nchk - 1, 0)


def _make_g2a(rw):
    nchk = rw // CH

    def body(i1h, i2h, t1h, t2h, outh,
             i1v0, i1v1, i2v0, i2v1, b10, b11, b20, b21,
             sa0, sa1, sb0, sb1, so0, so1):
        _, _, wid = _ids()
        i1v, i2v = (i1v0, i1v1), (i2v0, i2v1)
        b1v, b2v = (b10, b11), (b20, b21)
        sa, sb, so = (sa0, sa1), (sb0, sb1), (so0, so1)

        def drain(j, b):
            base = wid * rw + j * CH
            pltpu.make_async_copy(
                b1v[b], outh.at[pl.ds(base, CH)], so[b]).wait()

        def prime(j, b):
            base = wid * rw + j * CH

            @pl.when(j >= 2)
            def _():
                drain(j - 2, b)
            pltpu.sync_copy(i1h.at[pl.ds(base, CH)], i1v[b])
            pltpu.sync_copy(i2h.at[pl.ds(base, CH)], i2v[b])
            pltpu.async_copy(t1h.at[i1v[b]], b1v[b], sa[b])
            pltpu.async_copy(t2h.at[i2v[b]], b2v[b], sb[b])

        def work(j, b):
            base = wid * rw + j * CH
            pltpu.make_async_copy(t1h.at[i1v[b]], b1v[b], sa[b]).wait()
            pltpu.make_async_copy(t2h.at[i2v[b]], b2v[b], sb[b]).wait()

            @plsc.parallel_loop(0, CH, 1, unroll=4)
            def row(r):
                for q in range(D // 16):
                    sl = pl.ds(q * 16, 16)
                    b1v[b][r, sl] = b1v[b][r, sl] + b2v[b][r, sl]
            pltpu.async_copy(b1v[b], outh.at[pl.ds(base, CH)], so[b])

        _pipe(nchk, prime, work)
        drain(nchk - 2, (nchk - 2) % 2)
        drain(nchk - 1, (nchk - 1) % 2)

    return pl.kernel(
        body,
        out_type=jax.ShapeDtypeStruct((rw * NW, D), F32),
        mesh=_MESH,
        scratch_types=(
            [pltpu.VMEM((CH,), jnp.int32)] * 4
            + [pltpu.VMEM((CH, D), F32)] * 4
            + [pltpu.SemaphoreType.DMA] * 6
        ),
    )


def _make_g3(rw):
    nchk = rw // CH

    def body(ibh, idh, qth, kth, vth, oqh, okh, ovh,
             ib0, ib1, id0, id1, bq0, bq1, bk0, bk1, bv0, bv1,
             sq0, sq1, sk0, sk1, sv0, sv1, so0, so1):
        _, _, wid = _ids()
        ibv, idv = (ib0, ib1), (id0, id1)
        bq, bk, bv = (bq0, bq1), (bk0, bk1), (bv0, bv1)
        sq, sk, sv = (sq0, sq1), (sk0, sk1), (sv0, sv1)
        so = (so0, so1)

        def drain(j, b):
            base = wid * rw + j * CH
            pltpu.make_async_copy(bq[b], oqh.at[pl.ds(base, CH)], so[b]).wait()
            pltpu.make_async_copy(bk[b], okh.at[pl.ds(base, CH)], so[b]).wait()
            pltpu.make_async_copy(bv[b], ovh.at[pl.ds(base, CH)], so[b]).wait()

        def prime(j, b):
            base = wid * rw + j * CH

            @pl.when(j >= 2)
            def _():
                drain(j - 2, b)
            pltpu.sync_copy(ibh.at[pl.ds(base, CH)], ibv[b])
            pltpu.sync_copy(idh.at[pl.ds(base, CH)], idv[b])
            pltpu.async_copy(qth.at[ibv[b]], bq[b], sq[b])
            pltpu.async_copy(kth.at[idv[b]], bk[b], sk[b])
            pltpu.async_copy(vth.at[idv[b]], bv[b], sv[b])

        def work(j, b):
            base = wid * rw + j * CH
            pltpu.make_async_copy(qth.at[ibv[b]], bq[b], sq[b]).wait()
            pltpu.make_async_copy(kth.at[idv[b]], bk[b], sk[b]).wait()
            pltpu.make_async_copy(vth.at[idv[b]], bv[b], sv[b]).wait()
            pltpu.async_copy(bq[b], oqh.at[pl.ds(base, CH)], so[b])
            pltpu.async_copy(bk[b], okh.at[pl.ds(base, CH)], so[b])
            pltpu.async_copy(bv[b], ovh.at[pl.ds(base, CH)], so[b])

        _pipe(nchk, prime, work)
        drain(nchk - 2, (nchk - 2) % 2)
        drain(nchk - 1, (nchk - 1) % 2)

    return pl.kernel(
        body,
        out_type=(jax.ShapeDtypeStruct((rw * NW, D), F32),
                  jax.ShapeDtypeStruct((rw * NW, D), F32),
                  jax.ShapeDtypeStruct((rw * NW, D), F32)),
        mesh=_MESH,
        scratch_types=(
            [pltpu.VMEM((CH,), jnp.int32)] * 4
            + [pltpu.VMEM((CH, D), F32)] * 6
            + [pltpu.SemaphoreType.DMA] * 8
        ),
    )


def _make_scat(rw):
    nchk = rw // CH

    def body(idxh, mh, outh, iv0, iv1, bm0, bm1, s0, s1, so0, so1, acc):
        cid, sid, wid = _ids()
        iv, bm, sm = (iv0, iv1), (bm0, bm1), (s0, s1)
        so = (so0, so1)
        _zero_accum(bm0, acc, sid)
        plsc.subcore_barrier()

        def drain(j, b):
            pltpu.make_async_copy(bm[b], acc.at[iv[b]], so[b]).wait()

        def prime(j, b):
            base = wid * rw + j * CH

            @pl.when(j >= 2)
            def _():
                drain(j - 2, b)
            pltpu.sync_copy(idxh.at[pl.ds(base, CH)], iv[b])
            pltpu.async_copy(mh.at[pl.ds(base, CH)], bm[b], sm[b])

        def work(j, b):
            base = wid * rw + j * CH
            pltpu.make_async_copy(mh.at[pl.ds(base, CH)], bm[b], sm[b]).wait()
            pltpu.async_copy(bm[b], acc.at[iv[b]], so[b], add=True)

        _pipe(nchk, prime, work)
        drain(nchk - 2, (nchk - 2) % 2)
        drain(nchk - 1, (nchk - 1) % 2)
        plsc.subcore_barrier()
        _dump_accum(acc, outh, cid, sid)

    return pl.kernel(
        body,
        out_type=jax.ShapeDtypeStruct((2, PN, D), F32),
        mesh=_MESH,
        scratch_types=[
            pltpu.VMEM((CH,), jnp.int32), pltpu.VMEM((CH,), jnp.int32),
            pltpu.VMEM((CH, D), F32), pltpu.VMEM((CH, D), F32),
            pltpu.SemaphoreType.DMA, pltpu.SemaphoreType.DMA,
            pltpu.SemaphoreType.DMA, pltpu.SemaphoreType.DMA,
            pltpu.VMEM_SHARED((PN, D), F32),
        ],
    )


EA_, EB_ = 192000, 128000
RWA, RWB = EA_ // NW, EB_ // NW
_g2a_a, _g2a_b = _make_g2a(RWA), _make_g2a(RWB)
_g3_a, _g3_b = _make_g3(RWA), _make_g3(RWB)
_scat_a, _scat_b = _make_scat(RWA), _make_scat(RWB)


def _make_s2():
    def body(bsh, bdh, xph, cph, outh,
             ib0, ib1, id0, id1, bx0, bx1, bc0, bc1, sa0, sa1, sb0, sb1,
             so0, so1, acc):
        cid, sid, wid = _ids()
        ibv, idv = (ib0, ib1), (id0, id1)
        bx, bc = (bx0, bx1), (bc0, bc1)
        sa, sb, so = (sa0, sa1), (sb0, sb1), (so0, so1)
        _zero_accum(bx0, acc, sid)
        plsc.subcore_barrier()

        def drain(j, b):
            pltpu.make_async_copy(bx[b], acc.at[idv[b]], so[b]).wait()

        def prime(j, b):
            base = wid * RW + j * CH

            @pl.when(j >= 2)
            def _():
                drain(j - 2, b)
            pltpu.sync_copy(bsh.at[pl.ds(base, CH)], ibv[b])
            pltpu.sync_copy(bdh.at[pl.ds(base, CH)], idv[b])
            pltpu.async_copy(xph.at[ibv[b]], bx[b], sa[b])
            pltpu.async_copy(cph.at[idv[b]], bc[b], sb[b])

        def work(j, b):
            pltpu.make_async_copy(xph.at[ibv[b]], bx[b], sa[b]).wait()
            pltpu.make_async_copy(cph.at[idv[b]], bc[b], sb[b]).wait()

            @plsc.parallel_loop(0, CH, 1, unroll=2)
            def row(r):
                for q in range(D // 16):
                    sl = pl.ds(q * 16, 16)
                    x = bx[b][r, sl] + bc[b][r, sl]
                    x = x * (1.0 / (1.0 + jnp.exp(-x)))
                    bx[b][r, sl] = x * (1.0 / (1.0 + jnp.exp(-x)))
            pltpu.async_copy(bx[b], acc.at[idv[b]], so[b], add=True)

        _pipe(NCHK, prime, work)
        drain(NCHK - 2, (NCHK - 2) % 2)
        drain(NCHK - 1, (NCHK - 1) % 2)
        plsc.subcore_barrier()
        _dump_accum(acc, outh, cid, sid)

    return pl.kernel(
        body,
        out_type=jax.ShapeDtypeStruct((2, PN, D), F32),
        mesh=_MESH,
        scratch_types=(
            [pltpu.VMEM((CH,), jnp.int32)] * 4
            + [pltpu.VMEM((CH, D), F32)] * 4
            + [pltpu.SemaphoreType.DMA] * 6
            + [pltpu.VMEM_SHARED((PN, D), F32)]
        ),
    )


_s2 = _make_s2()


_RB = 1000
_EB = 2000


def _nspec(cols=D):
    return pl.BlockSpec((_RB, cols), lambda i: (i, 0))


def _wspec(r, c):
    return pl.BlockSpec((r, c), lambda i: (0, 0))


def _espec(cols=D):
    return pl.BlockSpec((_EB, cols), lambda i: (i, 0))


def _pspec(core):
    return pl.BlockSpec((1, _RB, D), lambda i, c=core: (c, i, 0))


def _tc_nodes1(h, w1dT, w1sT, b1):
    def body(h_ref, wd_ref, ws_ref, b1_ref, a_ref, b_ref):
        hh = h_ref[...]
        a_ref[...] = jnp.dot(hh, wd_ref[...], preferred_element_type=F32)
        b_ref[...] = jnp.dot(hh, ws_ref[...], preferred_element_type=F32) + b1_ref[...]
    return pl.pallas_call(
        body, grid=(N // _RB,),
        in_specs=[_nspec(), _wspec(D, D), _wspec(D, D), _wspec(1, D)],
        out_specs=[_nspec(), _nspec()],
        out_shape=[jax.ShapeDtypeStruct((N, D), F32)] * 2,
    )(h, w1dT, w1sT, b1)


def _tc_edge_mlp(g, ea, w1eT, w2T, b2):
    esz = g.shape[0]
    if ea is None:
        def body(g_ref, w2_ref, b2_ref, m_ref):
            m = _silu(g_ref[...])
            m = _silu(jnp.dot(m, w2_ref[...], preferred_element_type=F32)
                      + b2_ref[...])
            m_ref[...] = _silu(m)
        return pl.pallas_call(
            body, grid=(esz // _EB,),
            in_specs=[_espec(), _wspec(D, D), _wspec(1, D)],
            out_specs=_espec(),
            out_shape=jax.ShapeDtypeStruct((esz, D), F32),
        )(g, w2T, b2)

    def body(g_ref, ea_ref, w1e_ref, w2_ref, b2_ref, m_ref):
        pre = g_ref[...] + jnp.dot(ea_ref[...], w1e_ref[...],
                                   preferred_element_type=F32)
        m = _silu(pre)
        m = _silu(jnp.dot(m, w2_ref[...], preferred_element_type=F32)
                  + b2_ref[...])
        m_ref[...] = _silu(m)
    return pl.pallas_call(
        body, grid=(esz // _EB,),
        in_specs=[_espec(), _espec(16), _wspec(16, D), _wspec(D, D),
                  _wspec(1, D)],
        out_specs=_espec(),
        out_shape=jax.ShapeDtypeStruct((esz, D), F32),
    )(g, ea, w1eT, w2T, b2)


def _tc_update(pa, pb, x, uwT, ub):
    def body(p0_ref, p1_ref, p2_ref, p3_ref, x_ref, uw_ref, ub_ref, o_ref):
        s = p0_ref[0] + p1_ref[0] + p2_ref[0] + p3_ref[0]
        o_ref[...] = (x_ref[...]
                      + jnp.dot(s, uw_ref[...], preferred_element_type=F32)
                      + ub_ref[...])
    return pl.pallas_call(
        body, grid=(N // _RB,),
        in_specs=[_pspec(0), _pspec(1), _pspec(0), _pspec(1),
                  _nspec(), _wspec(D, D), _wspec(1, D)],
        out_specs=_nspec(),
        out_shape=jax.ShapeDtypeStruct((N, D), F32),
    )(pa, pa, pb, pb, x, uwT, ub)


def _tc_nodes2(h1, c, mwxT, mwcT, mb, g1wT, g1b, g2w, g2b):
    def body(h1_ref, c_ref, wx_ref, wc_ref, mb_ref, g1_ref, g1b_ref,
             g2_ref, g2b_ref, xp_ref, cp_ref, a16_ref):
        cc = c_ref[...]
        xp_ref[...] = jnp.dot(h1_ref[...], wx_ref[...],
                              preferred_element_type=F32)
        cp_ref[...] = jnp.dot(cc, wc_ref[...],
                              preferred_element_type=F32) + mb_ref[...]
        t = _silu(jnp.dot(cc, g1_ref[...], preferred_element_type=F32)
                  + g1b_ref[...])
        a = _sig(jnp.sum(t * g2_ref[...], axis=1, keepdims=True)
                 + g2b_ref[...])
        a16_ref[...] = jnp.broadcast_to(a, (_RB, 16))
    return pl.pallas_call(
        body, grid=(N // _RB,),
        in_specs=[_nspec(), _nspec(), _wspec(D, D), _wspec(D, D),
                  _wspec(1, D), _wspec(D, D), _wspec(1, D), _wspec(1, D),
                  _wspec(1, 1)],
        out_specs=[_nspec(), _nspec(), _nspec(16)],
        out_shape=[jax.ShapeDtypeStruct((N, D), F32),
                   jax.ShapeDtypeStruct((N, D), F32),
                   jax.ShapeDtypeStruct((N, 16), F32)],
    )(h1, c, mwxT, mwcT, mb, g1wT, g1b, g2w, g2b)


def _tc_update_gated(parts, x, uwT, ub, a16):
    def body(p0_ref, p1_ref, x_ref, uw_ref, ub_ref, a_ref, o_ref):
        s = (p0_ref[0] + p1_ref[0]) * a_ref[:, 0:1]
        o_ref[...] = (x_ref[...]
                      + jnp.dot(s, uw_ref[...], preferred_element_type=F32)
                      + ub_ref[...])
    return pl.pallas_call(
        body, grid=(N // _RB,),
        in_specs=[_pspec(0), _pspec(1), _nspec(), _wspec(D, D),
                  _wspec(1, D), _nspec(16)],
        out_specs=_nspec(),
        out_shape=jax.ShapeDtypeStruct((N, D), F32),
    )(parts, parts, x, uwT, ub, a16)


def _tc_add3_gated(pa, pb, x, skip, g16):
    def body(p0_ref, p1_ref, p2_ref, p3_ref, x_ref, s_ref, g_ref, o_ref):
        ps = p0_ref[0] + p1_ref[0] + p2_ref[0] + p3_ref[0]
        o_ref[...] = ps * g_ref[:, 0:1] + x_ref[...] + s_ref[...]
    return pl.pallas_call(
        body, grid=(N // _RB,),
        in_specs=[_pspec(0), _pspec(1), _pspec(0), _pspec(1),
                  _nspec(), _nspec(), _nspec(16)],
        out_specs=_nspec(),
        out_shape=jax.ShapeDtypeStruct((N, D), F32),
    )(pa, pa, pb, pb, x, skip, g16)


def _tc_nodes3a(h1, qwT, qb, g1wT, g1b, g2w, g2b):
    def body(h1_ref, qw_ref, qb_ref, g1_ref, g1b_ref, g2_ref, g2b_ref,
             q_ref, g16_ref):
        hh = h1_ref[...]
        q_ref[...] = jnp.dot(hh, qw_ref[...], preferred_element_type=F32) + qb_ref[...]
        t = _silu(jnp.dot(hh, g1_ref[...], preferred_element_type=F32)
                  + g1b_ref[...])
        g = _sig(jnp.sum(t * g2_ref[...], axis=1, keepdims=True)
                 + g2b_ref[...])
        g16_ref[...] = jnp.broadcast_to(g, (_RB, 16))
    return pl.pallas_call(
        body, grid=(N // _RB,),
        in_specs=[_nspec(), _wspec(D, D), _wspec(1, D),
                  _wspec(D, D), _wspec(1, D), _wspec(1, D), _wspec(1, 1)],
        out_specs=[_nspec(), _nspec(16)],
        out_shape=[jax.ShapeDtypeStruct((N, D), F32),
                   jax.ShapeDtypeStruct((N, 16), F32)],
    )(h1, qwT, qb, g1wT, g1b, g2w, g2b)


def _tc_nodes3b(c1, kwT, kb, vwT, vb):
    def body(c1_ref, kw_ref, kb_ref, vw_ref, vb_ref, k_ref, v_ref):
        cc = c1_ref[...]
        k_ref[...] = jnp.dot(cc, kw_ref[...], preferred_element_type=F32) + kb_ref[...]
        v_ref[...] = jnp.dot(cc, vw_ref[...], preferred_element_type=F32) + vb_ref[...]
    return pl.pallas_call(
        body, grid=(N // _RB,),
        in_specs=[_nspec(), _wspec(D, D), _wspec(1, D), _wspec(D, D),
                  _wspec(1, D)],
        out_specs=[_nspec(), _nspec()],
        out_shape=[jax.ShapeDtypeStruct((N, D), F32),
                   jax.ShapeDtypeStruct((N, D), F32)],
    )(c1, kwT, kb, vwT, vb)


def _tc_attn(gq, gk, gv, owT, ob):
    inv = float(1.0 / np.sqrt(HD))
    esz = gq.shape[0]

    def body(q_ref, k_ref, v_ref, ow_ref, ob_ref, m_ref):
        p = q_ref[...] * k_ref[...]
        s = [jnp.sum(p[:, HD * h:HD * (h + 1)], axis=1, keepdims=True) * inv
             for h in range(NH)]
        mx = jnp.maximum(jnp.maximum(s[0], s[1]), jnp.maximum(s[2], s[3]))
        e = [jnp.exp(sh - mx) for sh in s]
        z = e[0] + e[1] + e[2] + e[3]
        w = jnp.concatenate(
            [jnp.broadcast_to(eh / z, (_EB, HD)) for eh in e], axis=1)
        attn = w * v_ref[...]
        out = jnp.dot(attn, ow_ref[...], preferred_element_type=F32) + ob_ref[...]
        m_ref[...] = _silu(out)
    return pl.pallas_call(
        body, grid=(esz // _EB,),
        in_specs=[_espec(), _espec(), _espec(), _wspec(D, D), _wspec(1, D)],
        out_specs=_espec(),
        out_shape=jax.ShapeDtypeStruct((esz, D), F32),
    )(gq, gk, gv, owT, ob)


def kernel(H_rna, C, edge_attr_rna, im_w1, im_b1, im_w2, im_b2, im_uw,
           im_ub, mc_g1w, mc_g1b, mc_g2w, mc_g2b, mc_mw, mc_mb, mc_uw,
           mc_ub, cm_qw, cm_qb, cm_kw, cm_kb, cm_vw, cm_vb, cm_ow, cm_ob,
           cm_g1w, cm_g1b, cm_g2w, cm_g2b, cc_w1, cc_b1, cc_w2, cc_b2,
           cc_uw, cc_ub, edge_index_rna, edge_index_belong,
           edge_index_cell):
    src, dst = edge_index_rna[0], edge_index_rna[1]
    bsrc, bdst = edge_index_belong[0], edge_index_belong[1]
    csrc, cdst = edge_index_cell[0], edge_index_cell[1]
    row = lambda b: b.reshape(1, -1)
    half = lambda v: (v[:EA_], v[EA_:])

    src_a, src_b = half(src)
    dst_a, dst_b = half(dst)
    ea_a, ea_b = edge_attr_rna[:EA_], edge_attr_rna[EA_:]
    A, B = _tc_nodes1(H_rna, im_w1[:, :D].T, im_w1[:, D:2 * D].T,
                      row(im_b1))
    w1eT, w2T, b2 = im_w1[:, 2 * D:].T, im_w2.T, row(im_b2)
    G1a = _g2a_a(dst_a, src_a, A, B)
    M1a = _tc_edge_mlp(G1a, ea_a, w1eT, w2T, b2)
    G1b = _g2a_b(dst_b, src_b, A, B)
    M1b = _tc_edge_mlp(G1b, ea_b, w1eT, w2T, b2)
    P1a = _scat_a(dst_a, M1a)
    P1b = _scat_b(dst_b, M1b)
    H1 = _tc_update(P1a, P1b, H_rna, im_uw.T, row(im_ub))

    Xp, Cp, A16 = _tc_nodes2(H1, C, mc_mw[:, :D].T, mc_mw[:, D:].T,
                             row(mc_mb), mc_g1w.T, row(mc_g1b), mc_g2w,
                             mc_g2b.reshape(1, 1))
    P2 = _s2(bsrc, bdst, Xp, Cp)
    Qn, G16 = _tc_nodes3a(H1, cm_qw.T, row(cm_qb), cm_g1w.T, row(cm_g1b),
                          cm_g2w, cm_g2b.reshape(1, 1))
    C1 = _tc_update_gated(P2, C, mc_uw.T, row(mc_ub), A16)

    bsrc_a, bsrc_b = half(bsrc)
    bdst_a, bdst_b = half(bdst)
    Kn, Vn = _tc_nodes3b(C1, cm_kw.T, row(cm_kb), cm_vw.T, row(cm_vb))
    owT, ob = cm_ow.T, row(cm_ob)
    GQa, GKa, GVa = _g3_a(bsrc_a, bdst_a, Qn, Kn, Vn)
    M3a = _tc_attn(GQa, GKa, GVa, owT, ob)
    GQb, GKb, GVb = _g3_b(bsrc_b, bdst_b, Qn, Kn, Vn)
    M3b = _tc_attn(GQb, GKb, GVb, owT, ob)
    P3a = _scat_a(bsrc_a, M3a)
    P3b = _scat_b(bsrc_b, M3b)
    Hout = _tc_add3_gated(P3a, P3b, H1, H_rna, G16)

    csrc_a, csrc_b = half(csrc)
    cdst_a, cdst_b = half(cdst)
    A2, B2 = _tc_nodes1(C1, cc_w1[:, :D].T, cc_w1[:, D:].T, row(cc_b1))
    cw2T, cb2 = cc_w2.T, row(cc_b2)
    G4a = _g2a_a(cdst_a, csrc_a, A2, B2)
    M4a = _tc_edge_mlp(G4a, None, None, cw2T, cb2)
    G4b = _g2a_b(cdst_b, csrc_b, A2, B2)
    M4b = _tc_edge_mlp(G4b, None, None, cw2T, cb2)
    P4a = _scat_a(cdst_a, M4a)
    P4b = _scat_b(cdst_b, M4b)
    Cout = _tc_update_skip(P4a, P4b, C1, cc_uw.T, row(cc_ub), C)

    return (Hout, Cout)


def _tc_update_skip(pa, pb, x, uwT, ub, skip):
    def body(p0_ref, p1_ref, p2_ref, p3_ref, x_ref, uw_ref, ub_ref,
             s_ref, o_ref):
        s = p0_ref[0] + p1_ref[0] + p2_ref[0] + p3_ref[0]
        o_ref[...] = (x_ref[...]
                      + jnp.dot(s, uw_ref[...], preferred_element_type=F32)
                      + ub_ref[...] + s_ref[...])
    return pl.pallas_call(
        body, grid=(N // _RB,),
        in_specs=[_pspec(0), _pspec(1), _pspec(0), _pspec(1), _nspec(),
                  _wspec(D, D), _wspec(1, D), _nspec()],
        out_specs=_nspec(),
        out_shape=jax.ShapeDtypeStruct((N, D), F32),
    )(pa, pa, pb, pb, x, uwT, ub, skip)
